# Initial kernel scaffold; baseline (speedup 1.0000x reference)
import jax
import jax.numpy as jnp
from jax.experimental import pallas as pl


def kernel(features, edge_index, edge_weight, node_id, alpha, W0, b0, W1, b1, Wout, bout):
    raise NotImplementedError("write your pallas kernel here")


# speedup vs baseline: 33.9267x; 33.9267x over previous
import functools

import jax
import jax.numpy as jnp
from jax import lax
from jax.experimental import pallas as pl
from jax.experimental.pallas import tpu as pltpu
from jax.experimental.pallas import tpu_sc as plsc

N = 10000
NPAD = 10240
E = 320000
EPAD = 327680
D = 128
F = 16
C = 10
GENE = 5000
ALP = 5008
NC, NS, L = 2, 16, 16
NW = NC * NS
EPW = EPAD // NW
CH = 1024
NCHUNK = EPW // CH
NG = CH // L
NSTR = CH // 128
RPT = NPAD // NS

_f32 = jnp.float32
_i32 = jnp.int32


def _k1_body(x_ref, w0_ref, w1_ref, wout_ref, b0_ref, b1_ref, bout_ref,
             z_ref, rb_ref):
    wc = jnp.dot(wout_ref[...], w1_ref[...], preferred_element_type=_f32)
    wd = jnp.dot(wc, w0_ref[...], preferred_element_type=_f32)
    wdp = jnp.concatenate([wd, jnp.zeros((F - C, D), _f32)], axis=0)
    z_ref[...] = jnp.dot(x_ref[...], wdp.T, preferred_element_type=_f32)
    r = jnp.dot(b0_ref[...], wc.T, preferred_element_type=_f32)
    bc = jnp.dot(b1_ref[...], wout_ref[...].T,
                 preferred_element_type=_f32) + bout_ref[...]
    rb = jnp.concatenate([r, bc], axis=0)
    rb_ref[...] = jnp.concatenate([rb, jnp.zeros((2, F - C), _f32)], axis=1)


def _pass1_body(src_h, dst2_h, ew_h, nid_h, al_h, z_h,
                s1p_h, c_h,
                nid_v, al_v, src_v, dst2_v, ew_v, c_v, zr_v,
                s1_sh, sem):
    cid = lax.axis_index("c")
    sid = lax.axis_index("s")
    w = cid * NS + sid
    lane = lax.iota(_i32, L)
    zeros16 = jnp.zeros((L,), _f32)

    pltpu.sync_copy(nid_h, nid_v)
    pltpu.sync_copy(al_h, al_v)

    def _z(i, _):
        zr_v[i, :] = zeros16
        return 0
    lax.fori_loop(0, RPT, _z, 0)
    pltpu.sync_copy(zr_v.at[pl.ds(0, RPT)], s1_sh.at[pl.ds(sid * RPT, RPT)])
    plsc.subcore_barrier()

    def _chunk(ch, _):
        base = pl.multiple_of(w * EPW + ch * CH, CH)
        row0 = pl.multiple_of(base // 128, NSTR)
        pltpu.sync_copy(src_h.at[pl.ds(base, CH)], src_v)
        pltpu.sync_copy(dst2_h.at[pl.ds(row0, NSTR)], dst2_v)
        pltpu.sync_copy(ew_h.at[pl.ds(base, CH)], ew_v)
        gath = [pltpu.async_copy(z_h.at[src_v.at[pl.ds(j * 128, 128)]],
                                 zr_v.at[pl.ds(j * 128, 128)], sem)
                for j in range(NSTR)]

        def _cg(g, _):
            s16 = src_v[pl.ds(g * L, L)]
            d16 = dst2_v[g // 8, pl.ds((g % 8) * L, L)]
            sid16 = plsc.load_gather(nid_v, [s16])
            did16 = plsc.load_gather(nid_v, [d16])
            sg = sid16 >= 0
            dg = did16 >= 0
            idx16 = jnp.full((L,), GENE + 1, _i32)
            idx16 = jnp.where(sg & (~dg), sid16, idx16)
            idx16 = jnp.where(dg & (~sg), did16, idx16)
            idx16 = jnp.where(dg & sg, jnp.full((L,), GENE, _i32), idx16)
            a16 = plsc.load_gather(al_v, [idx16])
            c_v[pl.ds(g * L, L)] = a16 * ew_v[pl.ds(g * L, L)]
            return 0
        lax.fori_loop(0, NG, _cg, 0)
        for g in gath:
            g.wait()

        def _eg(g, _):
            for e in range(L):
                i = g * L + e
                row = zr_v[i, :]
                ce = plsc.load_gather(c_v, [jnp.full((L,), i, _i32)])
                out = jnp.where(lane < 14, row * ce,
                                jnp.where(lane == 14, 1.0, ce))
                zr_v[i, :] = out
            return 0
        lax.fori_loop(0, NG, _eg, 0)

        scat = [pltpu.async_copy(zr_v.at[pl.ds(j * 128, 128)],
                                 s1_sh.at[dst2_v.at[j]], sem, add=True)
                for j in range(NSTR)]
        pltpu.sync_copy(c_v, c_h.at[pl.ds(base, CH)])
        for s in scat:
            s.wait()
        return 0
    lax.fori_loop(0, NCHUNK, _chunk, 0)

    plsc.subcore_barrier()
    pltpu.sync_copy(s1_sh.at[pl.ds(sid * RPT, RPT)],
                    s1p_h.at[cid, pl.ds(sid * RPT, RPT)])


def _k3_body(s1p_ref, az_ref, sc_ref, srn_ref):
    s = s1p_ref[0] + s1p_ref[1]
    cnt = s[:, 14:15]
    rs = s[:, 15:16]
    scale = 1.0 / jnp.maximum(cnt, 1.0)
    colmask = lax.broadcasted_iota(_i32, (NPAD, F), 1) < C
    az = jnp.where(colmask, s * scale, 0.0)
    az_ref[...] = az[:N]
    sc_ref[...] = scale[:N]
    srn_ref[...] = (scale * rs)[:N]


def _pass2_body(src_h, dst2_h, cin_h, az_h,
                s2p_h,
                src_v, dst2_v, c_v, zr_v, s2_sh, sem):
    cid = lax.axis_index("c")
    sid = lax.axis_index("s")
    w = cid * NS + sid
    zeros16 = jnp.zeros((L,), _f32)

    def _z(i, _):
        zr_v[i, :] = zeros16
        return 0
    lax.fori_loop(0, RPT, _z, 0)
    pltpu.sync_copy(zr_v.at[pl.ds(0, RPT)], s2_sh.at[pl.ds(sid * RPT, RPT)])
    plsc.subcore_barrier()

    def _chunk(ch, _):
        base = pl.multiple_of(w * EPW + ch * CH, CH)
        row0 = pl.multiple_of(base // 128, NSTR)
        pltpu.sync_copy(src_h.at[pl.ds(base, CH)], src_v)
        pltpu.sync_copy(dst2_h.at[pl.ds(row0, NSTR)], dst2_v)
        pltpu.sync_copy(cin_h.at[pl.ds(base, CH)], c_v)
        gath = [pltpu.async_copy(az_h.at[src_v.at[pl.ds(j * 128, 128)]],
                                 zr_v.at[pl.ds(j * 128, 128)], sem)
                for j in range(NSTR)]
        for g in gath:
            g.wait()

        def _eg(g, _):
            for e in range(L):
                i = g * L + e
                row = zr_v[i, :]
                ce = plsc.load_gather(c_v, [jnp.full((L,), i, _i32)])
                zr_v[i, :] = row * ce
            return 0
        lax.fori_loop(0, NG, _eg, 0)

        scat = [pltpu.async_copy(zr_v.at[pl.ds(j * 128, 128)],
                                 s2_sh.at[dst2_v.at[j]], sem, add=True)
                for j in range(NSTR)]
        for s in scat:
            s.wait()
        return 0
    lax.fori_loop(0, NCHUNK, _chunk, 0)

    plsc.subcore_barrier()
    pltpu.sync_copy(s2_sh.at[pl.ds(sid * RPT, RPT)],
                    s2p_h.at[cid, pl.ds(sid * RPT, RPT)])


def _k5_body(s2p_ref, sc_ref, srn_ref, rb_ref, out_ref):
    s2 = (s2p_ref[0] + s2p_ref[1])[:N, :C]
    r = rb_ref[0:1, :C]
    bc = rb_ref[1:2, :C]
    out_ref[...] = sc_ref[...] * s2 + srn_ref[...] * r + bc


def kernel(features, edge_index, edge_weight, node_id, alpha,
           W0, b0, W1, b1, Wout, bout):
    src = edge_index[0]
    dst = edge_index[1]
    ew = edge_weight[:, 0]
    pad = EPAD - E
    pad_ar = jnp.arange(pad, dtype=_i32)
    src_p = jnp.concatenate([src, pad_ar % N])
    dst_p = jnp.concatenate([dst, N + pad_ar % (NPAD - N)])
    ew_p = jnp.concatenate([ew, jnp.zeros((pad,), _f32)])
    dst2d = dst_p.reshape(EPAD // 128, 128)
    al_p = jnp.concatenate([alpha[:, 0], jnp.zeros((ALP - GENE - 2,), _f32)])

    z, rb = pl.pallas_call(
        _k1_body,
        out_shape=[jax.ShapeDtypeStruct((N, F), _f32),
                   jax.ShapeDtypeStruct((2, F), _f32)],
    )(features, W0, W1, Wout, b0[None, :], b1[None, :], bout[None, :])

    mesh = plsc.VectorSubcoreMesh(core_axis_name="c", subcore_axis_name="s")

    pass1 = pl.kernel(
        _pass1_body,
        out_type=[jax.ShapeDtypeStruct((NC, NPAD, F), _f32),
                  jax.ShapeDtypeStruct((EPAD,), _f32)],
        mesh=mesh,
        compiler_params=pltpu.CompilerParams(needs_layout_passes=False, use_tc_tiling_on_sc=False),
        scratch_types=[pltpu.VMEM((N,), _i32),
                       pltpu.VMEM((ALP,), _f32),
                       pltpu.VMEM((CH,), _i32),
                       pltpu.VMEM((NSTR, 128), _i32),
                       pltpu.VMEM((CH,), _f32),
                       pltpu.VMEM((CH,), _f32),
                       pltpu.VMEM((CH, F), _f32),
                       pltpu.VMEM_SHARED((NPAD, F), _f32),
                       pltpu.SemaphoreType.DMA],
    )
    s1p, cedge = pass1(src_p, dst2d, ew_p, node_id, al_p, z)

    az, scl, srn = pl.pallas_call(
        _k3_body,
        out_shape=[jax.ShapeDtypeStruct((N, F), _f32),
                   jax.ShapeDtypeStruct((N, 1), _f32),
                   jax.ShapeDtypeStruct((N, 1), _f32)],
    )(s1p)

    pass2 = pl.kernel(
        _pass2_body,
        out_type=[jax.ShapeDtypeStruct((NC, NPAD, F), _f32)],
        mesh=mesh,
        compiler_params=pltpu.CompilerParams(needs_layout_passes=False, use_tc_tiling_on_sc=False),
        scratch_types=[pltpu.VMEM((CH,), _i32),
                       pltpu.VMEM((NSTR, 128), _i32),
                       pltpu.VMEM((CH,), _f32),
                       pltpu.VMEM((CH, F), _f32),
                       pltpu.VMEM_SHARED((NPAD, F), _f32),
                       pltpu.SemaphoreType.DMA],
    )
    (s2p,) = pass2(src_p, dst2d, cedge, az)

    out = pl.pallas_call(
        _k5_body,
        out_shape=jax.ShapeDtypeStruct((N, C), _f32),
    )(s2p, scl, srn, rb)
    return out


# speedup vs baseline: 40.1897x; 1.1846x over previous
import jax
import jax.numpy as jnp
from jax import lax
from jax.experimental import pallas as pl
from jax.experimental.pallas import tpu as pltpu
from jax.experimental.pallas import tpu_sc as plsc

N = 10000
NPAD = 10240
E = 320000
EPAD = 327680
D = 128
F = 16
C = 10
GENE = 5000
ALP = 5008
NC, NS, L = 2, 16, 16
NW = NC * NS
EPW = EPAD // NW
CH = 1024
NCHUNK = EPW // CH
NG = CH // L
NSTR = CH // 128
RPT = NPAD // NS

_f32 = jnp.float32
_i32 = jnp.int32


def _k1_body(x_ref, w0_ref, w1_ref, wout_ref, b0_ref, b1_ref, bout_ref,
             z_ref, rb_ref):
    wc = jnp.dot(wout_ref[...], w1_ref[...], preferred_element_type=_f32)
    wd = jnp.dot(wc, w0_ref[...], preferred_element_type=_f32)
    wdp = jnp.concatenate([wd, jnp.zeros((F - C, D), _f32)], axis=0)
    z_ref[...] = jnp.dot(x_ref[...], wdp.T, preferred_element_type=_f32)
    r = jnp.dot(b0_ref[...], wc.T, preferred_element_type=_f32)
    bc = jnp.dot(b1_ref[...], wout_ref[...].T,
                 preferred_element_type=_f32) + bout_ref[...]
    rb = jnp.concatenate([r, bc], axis=0)
    rb_ref[...] = jnp.concatenate([rb, jnp.zeros((2, F - C), _f32)], axis=1)


def _zero_shared(zr, sh, sid):
    zeros16 = jnp.zeros((L,), _f32)

    def _z(i, _):
        zr[i, :] = zeros16
        return 0
    lax.fori_loop(0, RPT, _z, 0)
    pltpu.sync_copy(zr.at[pl.ds(0, RPT)], sh.at[pl.ds(sid * RPT, RPT)])
    plsc.subcore_barrier()


def _pass1_body(src_h, dst2_h, ew_h, nid_h, al_h, z_h,
                s1p_h, c_h,
                nid_v, al_v, src_v, dst2_v, ew_v, c_v, zr_v,
                s1_sh, lsem, gsem, ssem, wsem):
    cid = lax.axis_index("c")
    sid = lax.axis_index("s")
    w = cid * NS + sid
    lane = lax.iota(_i32, L)
    ones16 = jnp.ones((L,), _f32)

    pltpu.sync_copy(nid_h, nid_v)
    pltpu.sync_copy(al_h, al_v)
    _zero_shared(zr_v[0], s1_sh, sid)

    def fire_loads(k):
        b = k % 3
        base = pl.multiple_of(w * EPW + k * CH, CH)
        row0 = pl.multiple_of(base // 128, NSTR)
        return [pltpu.async_copy(src_h.at[pl.ds(base, CH)], src_v[b], lsem[b]),
                pltpu.async_copy(dst2_h.at[pl.ds(row0, NSTR)], dst2_v[b],
                                 lsem[b]),
                pltpu.async_copy(ew_h.at[pl.ds(base, CH)], ew_v[b], lsem[b])]

    def fire_gathers(k):
        b, p = k % 3, k % 2
        return [pltpu.async_copy(z_h.at[src_v[b].at[pl.ds(j * 128, 128)]],
                                 zr_v[p].at[pl.ds(j * 128, 128)], gsem[p])
                for j in range(NSTR)]

    def compute_c(k):
        b = k % 3

        def _cg(g, _):
            s16 = src_v[b][pl.ds(g * L, L)]
            d16 = dst2_v[b][g // 8, pl.ds((g % 8) * L, L)]
            sid16 = plsc.load_gather(nid_v, [s16])
            did16 = plsc.load_gather(nid_v, [d16])
            sg = sid16 >= 0
            dg = did16 >= 0
            idx16 = jnp.full((L,), GENE + 1, _i32)
            idx16 = jnp.where(sg & (~dg), sid16, idx16)
            idx16 = jnp.where(dg & (~sg), did16, idx16)
            idx16 = jnp.where(dg & sg, jnp.full((L,), GENE, _i32), idx16)
            a16 = plsc.load_gather(al_v, [idx16])
            c_v[b][pl.ds(g * L, L)] = a16 * ew_v[b][pl.ds(g * L, L)]
            return 0
        lax.fori_loop(0, NG, _cg, 0)
        base = pl.multiple_of(w * EPW + k * CH, CH)
        return pltpu.async_copy(c_v[b], c_h.at[pl.ds(base, CH)], wsem[k % 2])

    def scale_rows(k):
        b, p = k % 3, k % 2

        def _sg(g, _):
            rowi = g * L + lane
            c16 = c_v[b][pl.ds(g * L, L)]
            for j in range(C):
                cj = jnp.full((L,), j, _i32)
                col = plsc.load_gather(zr_v[p], [rowi, cj])
                plsc.store_scatter(zr_v[p], [rowi, cj], col * c16)
            plsc.store_scatter(zr_v[p], [rowi, jnp.full((L,), 14, _i32)],
                               ones16)
            plsc.store_scatter(zr_v[p], [rowi, jnp.full((L,), 15, _i32)], c16)
            return 0
        lax.fori_loop(0, NG, _sg, 0)

    def fire_scatters(k):
        b, p = k % 3, k % 2
        return [pltpu.async_copy(zr_v[p].at[pl.ds(j * 128, 128)],
                                 s1_sh.at[dst2_v[b].at[j]], ssem[p], add=True)
                for j in range(NSTR)]

    loads = {0: fire_loads(0)}
    gath, scat, cw = {}, {}, {}
    for k in range(NCHUNK + 1):
        if k < NCHUNK:
            if k >= 2:
                for s in scat[k - 2]:
                    s.wait()
                cw[k - 2].wait()
            for d in loads[k]:
                d.wait()
            gath[k] = fire_gathers(k)
            if k + 1 < NCHUNK:
                loads[k + 1] = fire_loads(k + 1)
            cw[k] = compute_c(k)
        if k >= 1:
            for g in gath[k - 1]:
                g.wait()
            scale_rows(k - 1)
            scat[k - 1] = fire_scatters(k - 1)
    for s in scat[NCHUNK - 2]:
        s.wait()
    for s in scat[NCHUNK - 1]:
        s.wait()
    cw[NCHUNK - 1].wait()
    cw[NCHUNK - 2].wait()

    plsc.subcore_barrier()
    pltpu.sync_copy(s1_sh.at[pl.ds(sid * RPT, RPT)],
                    s1p_h.at[cid, pl.ds(sid * RPT, RPT)])


def _k3_body(s1p_ref, az_ref, sc_ref, srn_ref):
    s = s1p_ref[0] + s1p_ref[1]
    cnt = s[:, 14:15]
    rs = s[:, 15:16]
    scale = 1.0 / jnp.maximum(cnt, 1.0)
    colmask = lax.broadcasted_iota(_i32, (NPAD, F), 1) < C
    az = jnp.where(colmask, s * scale, 0.0)
    az_ref[...] = az[:N]
    sc_ref[...] = scale[:N]
    srn_ref[...] = (scale * rs)[:N]


def _pass2_body(src_h, dst2_h, cin_h, az_h,
                s2p_h,
                src_v, dst2_v, c_v, zr_v, s2_sh, lsem, gsem, ssem):
    cid = lax.axis_index("c")
    sid = lax.axis_index("s")
    w = cid * NS + sid
    lane = lax.iota(_i32, L)

    _zero_shared(zr_v[0], s2_sh, sid)

    def fire_loads(k):
        b = k % 3
        base = pl.multiple_of(w * EPW + k * CH, CH)
        row0 = pl.multiple_of(base // 128, NSTR)
        return [pltpu.async_copy(src_h.at[pl.ds(base, CH)], src_v[b], lsem[b]),
                pltpu.async_copy(dst2_h.at[pl.ds(row0, NSTR)], dst2_v[b],
                                 lsem[b]),
                pltpu.async_copy(cin_h.at[pl.ds(base, CH)], c_v[b], lsem[b])]

    def fire_gathers(k):
        b, p = k % 3, k % 2
        return [pltpu.async_copy(az_h.at[src_v[b].at[pl.ds(j * 128, 128)]],
                                 zr_v[p].at[pl.ds(j * 128, 128)], gsem[p])
                for j in range(NSTR)]

    def scale_rows(k):
        b, p = k % 3, k % 2

        def _sg(g, _):
            rowi = g * L + lane
            c16 = c_v[b][pl.ds(g * L, L)]
            for j in range(C):
                cj = jnp.full((L,), j, _i32)
                col = plsc.load_gather(zr_v[p], [rowi, cj])
                plsc.store_scatter(zr_v[p], [rowi, cj], col * c16)
            return 0
        lax.fori_loop(0, NG, _sg, 0)

    def fire_scatters(k):
        b, p = k % 3, k % 2
        return [pltpu.async_copy(zr_v[p].at[pl.ds(j * 128, 128)],
                                 s2_sh.at[dst2_v[b].at[j]], ssem[p], add=True)
                for j in range(NSTR)]

    loads = {0: fire_loads(0)}
    gath, scat = {}, {}
    for k in range(NCHUNK + 1):
        if k < NCHUNK:
            if k >= 2:
                for s in scat[k - 2]:
                    s.wait()
            for d in loads[k]:
                d.wait()
            gath[k] = fire_gathers(k)
            if k + 1 < NCHUNK:
                loads[k + 1] = fire_loads(k + 1)
        if k >= 1:
            for g in gath[k - 1]:
                g.wait()
            scale_rows(k - 1)
            scat[k - 1] = fire_scatters(k - 1)
    for s in scat[NCHUNK - 2]:
        s.wait()
    for s in scat[NCHUNK - 1]:
        s.wait()

    plsc.subcore_barrier()
    pltpu.sync_copy(s2_sh.at[pl.ds(sid * RPT, RPT)],
                    s2p_h.at[cid, pl.ds(sid * RPT, RPT)])


def _k5_body(s2p_ref, sc_ref, srn_ref, rb_ref, out_ref):
    s2 = (s2p_ref[0] + s2p_ref[1])[:N, :C]
    r = rb_ref[0:1, :C]
    bc = rb_ref[1:2, :C]
    out_ref[...] = sc_ref[...] * s2 + srn_ref[...] * r + bc


def kernel(features, edge_index, edge_weight, node_id, alpha,
           W0, b0, W1, b1, Wout, bout):
    src = edge_index[0]
    dst = edge_index[1]
    ew = edge_weight[:, 0]
    pad = EPAD - E
    pad_ar = jnp.arange(pad, dtype=_i32)
    src_p = jnp.concatenate([src, pad_ar % N])
    dst_p = jnp.concatenate([dst, N + pad_ar % (NPAD - N)])
    ew_p = jnp.concatenate([ew, jnp.zeros((pad,), _f32)])
    dst2d = dst_p.reshape(EPAD // 128, 128)
    al_p = jnp.concatenate([alpha[:, 0], jnp.zeros((ALP - GENE - 2,), _f32)])

    z, rb = pl.pallas_call(
        _k1_body,
        out_shape=[jax.ShapeDtypeStruct((N, F), _f32),
                   jax.ShapeDtypeStruct((2, F), _f32)],
    )(features, W0, W1, Wout, b0[None, :], b1[None, :], bout[None, :])

    mesh = plsc.VectorSubcoreMesh(core_axis_name="c", subcore_axis_name="s")
    params = pltpu.CompilerParams(needs_layout_passes=False,
                                  use_tc_tiling_on_sc=False)

    pass1 = pl.kernel(
        _pass1_body,
        out_type=[jax.ShapeDtypeStruct((NC, NPAD, F), _f32),
                  jax.ShapeDtypeStruct((EPAD,), _f32)],
        mesh=mesh,
        compiler_params=params,
        scratch_types=[pltpu.VMEM((N,), _i32),
                       pltpu.VMEM((ALP,), _f32),
                       [pltpu.VMEM((CH,), _i32)] * 3,
                       [pltpu.VMEM((NSTR, 128), _i32)] * 3,
                       [pltpu.VMEM((CH,), _f32)] * 3,
                       [pltpu.VMEM((CH,), _f32)] * 3,
                       [pltpu.VMEM((CH, F), _f32)] * 2,
                       pltpu.VMEM_SHARED((NPAD, F), _f32),
                       [pltpu.SemaphoreType.DMA] * 3,
                       [pltpu.SemaphoreType.DMA] * 2,
                       [pltpu.SemaphoreType.DMA] * 2,
                       [pltpu.SemaphoreType.DMA] * 2],
    )
    s1p, cedge = pass1(src_p, dst2d, ew_p, node_id, al_p, z)

    az, scl, srn = pl.pallas_call(
        _k3_body,
        out_shape=[jax.ShapeDtypeStruct((N, F), _f32),
                   jax.ShapeDtypeStruct((N, 1), _f32),
                   jax.ShapeDtypeStruct((N, 1), _f32)],
    )(s1p)

    pass2 = pl.kernel(
        _pass2_body,
        out_type=[jax.ShapeDtypeStruct((NC, NPAD, F), _f32)],
        mesh=mesh,
        compiler_params=params,
        scratch_types=[[pltpu.VMEM((CH,), _i32)] * 3,
                       [pltpu.VMEM((NSTR, 128), _i32)] * 3,
                       [pltpu.VMEM((CH,), _f32)] * 3,
                       [pltpu.VMEM((CH, F), _f32)] * 2,
                       pltpu.VMEM_SHARED((NPAD, F), _f32),
                       [pltpu.SemaphoreType.DMA] * 3,
                       [pltpu.SemaphoreType.DMA] * 2,
                       [pltpu.SemaphoreType.DMA] * 2],
    )
    (s2p,) = pass2(src_p, dst2d, cedge, az)

    out = pl.pallas_call(
        _k5_body,
        out_shape=jax.ShapeDtypeStruct((N, C), _f32),
    )(s2p, scl, srn, rb)
    return out


# speedup vs baseline: 56.8191x; 1.4138x over previous
import jax
import jax.numpy as jnp
from jax import lax
from jax.experimental import pallas as pl
from jax.experimental.pallas import tpu as pltpu
from jax.experimental.pallas import tpu_sc as plsc

N = 10000
NPAD = 10240
E = 320000
EPAD = 327680
D = 128
C = 10
NP = 5
F1 = 16
F2 = 16
GENE = 5000
ALP = 5008
NC, NS, L = 2, 16, 16
NW = NC * NS
EPW = EPAD // NW
CH = 1024
NCHUNK = EPW // CH
NG = CH // L
NSTR = CH // 128
RPT = NPAD // NS

_f32 = jnp.float32
_i32 = jnp.int32


def _pack_bf16_pairs_t(xt):
    b = lax.bitcast_convert_type(xt, _i32)
    r = b + jnp.int32(0x7FFF) + (lax.shift_right_logical(b, 16) & 1)
    h = lax.shift_right_logical(r, 16)
    rows = [h[2 * j:2 * j + 1, :]
            | lax.shift_left(h[2 * j + 1:2 * j + 2, :], 16)
            for j in range(NP)]
    return jnp.concatenate(rows, axis=0)


def _k1_body(x_ref, w0_ref, w1_ref, wout_ref, b0_ref, b1_ref, bout_ref,
             zp_ref, rb_ref):
    wc = jnp.dot(wout_ref[...], w1_ref[...], preferred_element_type=_f32)
    wd = jnp.dot(wc, w0_ref[...], preferred_element_type=_f32)
    zt = lax.dot_general(wd, x_ref[...], (((1,), (1,)), ((), ())),
                         preferred_element_type=_f32)
    zp_ref[...] = _pack_bf16_pairs_t(zt)
    r = jnp.dot(b0_ref[...], wc.T, preferred_element_type=_f32)
    bc = jnp.dot(b1_ref[...], wout_ref[...].T,
                 preferred_element_type=_f32) + bout_ref[...]
    rb_ref[...] = jnp.concatenate([r, bc], axis=0)


def _zero_acc(zs, sh, sid, width):
    lane = lax.iota(_i32, L)
    zeros16 = jnp.zeros((L,), _f32)

    def _z(g, _):
        rowi = g * L + lane
        for j in range(width):
            plsc.store_scatter(zs, [rowi, jnp.full((L,), j, _i32)], zeros16)
        return 0
    lax.fori_loop(0, RPT // L, _z, 0)
    pltpu.sync_copy(zs.at[pl.ds(0, RPT)], sh.at[pl.ds(sid * RPT, RPT)])
    plsc.subcore_barrier()


def _unpack_cols(w):
    lo = plsc.bitcast(lax.shift_left(w, 16), _f32)
    hi = plsc.bitcast(w & jnp.int32(-65536), _f32)
    return lo, hi


def _pass1_body(src_h, dst2_h, ew_h, nid_h, al_h, zp_h,
                s1p_h, c_h,
                nid_v, al_v, zp_v, src_v, dst2_v, ew_v, c_v, zs_v,
                s1_sh, lsem, ssem, wsem):
    cid = lax.axis_index("c")
    sid = lax.axis_index("s")
    w = cid * NS + sid
    lane = lax.iota(_i32, L)
    ones16 = jnp.ones((L,), _f32)

    pltpu.sync_copy(nid_h, nid_v)
    pltpu.sync_copy(al_h, al_v)
    for j in range(NP):
        pltpu.sync_copy(zp_h.at[j], zp_v.at[pl.ds(j * N, N)])
    _zero_acc(zs_v[0], s1_sh, sid, F1)

    def fire_loads(k):
        b = k % 3
        base = pl.multiple_of(w * EPW + k * CH, CH)
        row0 = pl.multiple_of(base // 128, NSTR)
        return [pltpu.async_copy(src_h.at[pl.ds(base, CH)], src_v[b], lsem[b]),
                pltpu.async_copy(dst2_h.at[pl.ds(row0, NSTR)], dst2_v[b],
                                 lsem[b]),
                pltpu.async_copy(ew_h.at[pl.ds(base, CH)], ew_v[b], lsem[b])]

    def compute(k):
        b, p = k % 3, k % 2

        def _g(g, _):
            s16 = src_v[b][pl.ds(g * L, L)]
            d16 = dst2_v[b][g // 8, pl.ds((g % 8) * L, L)]
            sid16 = plsc.load_gather(nid_v, [s16])
            did16 = plsc.load_gather(nid_v, [d16])
            sg = sid16 >= 0
            dg = did16 >= 0
            idx16 = jnp.full((L,), GENE + 1, _i32)
            idx16 = jnp.where(sg & (~dg), sid16, idx16)
            idx16 = jnp.where(dg & (~sg), did16, idx16)
            idx16 = jnp.where(dg & sg, jnp.full((L,), GENE, _i32), idx16)
            a16 = plsc.load_gather(al_v, [idx16])
            c16 = a16 * ew_v[b][pl.ds(g * L, L)]
            c_v[b][pl.ds(g * L, L)] = c16
            rowi = g * L + lane
            for jp in range(NP):
                wrd = plsc.load_gather(zp_v, [s16 + (jp * N)])
                lo, hi = _unpack_cols(wrd)
                plsc.store_scatter(zs_v[p], [rowi, jnp.full((L,), 2 * jp,
                                                            _i32)], lo * c16)
                plsc.store_scatter(zs_v[p], [rowi, jnp.full((L,), 2 * jp + 1,
                                                            _i32)], hi * c16)
            plsc.store_scatter(zs_v[p], [rowi, jnp.full((L,), 14, _i32)],
                               ones16)
            plsc.store_scatter(zs_v[p], [rowi, jnp.full((L,), 15, _i32)], c16)
            return 0
        lax.fori_loop(0, NG, _g, 0)
        base = pl.multiple_of(w * EPW + k * CH, CH)
        return pltpu.async_copy(c_v[b], c_h.at[pl.ds(base, CH)], wsem[b])

    def fire_scatters(k):
        b, p = k % 3, k % 2
        return [pltpu.async_copy(zs_v[p].at[pl.ds(j * 128, 128)],
                                 s1_sh.at[dst2_v[b].at[j]], ssem[p], add=True)
                for j in range(NSTR)]

    loads = {0: fire_loads(0)}
    scat, cw = {}, {}
    for k in range(NCHUNK):
        if k >= 2:
            for s in scat[k - 2]:
                s.wait()
        if k >= 3:
            cw[k - 3].wait()
        for d in loads[k]:
            d.wait()
        if k + 1 < NCHUNK:
            loads[k + 1] = fire_loads(k + 1)
        cw[k] = compute(k)
        scat[k] = fire_scatters(k)
    for s in scat[NCHUNK - 2]:
        s.wait()
    for s in scat[NCHUNK - 1]:
        s.wait()
    for k in range(NCHUNK - 3, NCHUNK):
        cw[k].wait()

    plsc.subcore_barrier()
    pltpu.sync_copy(s1_sh.at[pl.ds(sid * RPT, RPT)],
                    s1p_h.at[cid, pl.ds(sid * RPT, RPT)])


def _k3_body(s1p_ref, azp_ref, sc_ref, srn_ref):
    s = s1p_ref[0] + s1p_ref[1]
    cnt = s[:, 14:15]
    rs = s[:, 15:16]
    scale = 1.0 / jnp.maximum(cnt, 1.0)
    az = (s[:, :C] * scale)[:N]
    azp_ref[...] = _pack_bf16_pairs_t(az.T)
    sc_ref[...] = scale[:N]
    srn_ref[...] = (scale * rs)[:N]


def _pass2_body(src_h, dst2_h, cin_h, azp_h,
                s2p_h,
                azp_v, src_v, dst2_v, c_v, zs_v, lsem, ssem, s2_sh):
    cid = lax.axis_index("c")
    sid = lax.axis_index("s")
    w = cid * NS + sid
    lane = lax.iota(_i32, L)

    for j in range(NP):
        pltpu.sync_copy(azp_h.at[j], azp_v.at[pl.ds(j * N, N)])
    _zero_acc(zs_v[0], s2_sh, sid, F2)

    def fire_loads(k):
        b = k % 3
        base = pl.multiple_of(w * EPW + k * CH, CH)
        row0 = pl.multiple_of(base // 128, NSTR)
        return [pltpu.async_copy(src_h.at[pl.ds(base, CH)], src_v[b], lsem[b]),
                pltpu.async_copy(dst2_h.at[pl.ds(row0, NSTR)], dst2_v[b],
                                 lsem[b]),
                pltpu.async_copy(cin_h.at[pl.ds(base, CH)], c_v[b], lsem[b])]

    def compute(k):
        b, p = k % 3, k % 2

        def _g(g, _):
            s16 = src_v[b][pl.ds(g * L, L)]
            c16 = c_v[b][pl.ds(g * L, L)]
            rowi = g * L + lane
            for jp in range(NP):
                wrd = plsc.load_gather(azp_v, [s16 + (jp * N)])
                lo, hi = _unpack_cols(wrd)
                plsc.store_scatter(zs_v[p], [rowi, jnp.full((L,), 2 * jp,
                                                            _i32)], lo * c16)
                plsc.store_scatter(zs_v[p], [rowi, jnp.full((L,), 2 * jp + 1,
                                                            _i32)], hi * c16)
            return 0
        lax.fori_loop(0, NG, _g, 0)

    def fire_scatters(k):
        b, p = k % 3, k % 2
        return [pltpu.async_copy(zs_v[p].at[pl.ds(j * 128, 128)],
                                 s2_sh.at[dst2_v[b].at[j]], ssem[p], add=True)
                for j in range(NSTR)]

    loads = {0: fire_loads(0)}
    scat = {}
    for k in range(NCHUNK):
        if k >= 2:
            for s in scat[k - 2]:
                s.wait()
        for d in loads[k]:
            d.wait()
        if k + 1 < NCHUNK:
            loads[k + 1] = fire_loads(k + 1)
        compute(k)
        scat[k] = fire_scatters(k)
    for s in scat[NCHUNK - 2]:
        s.wait()
    for s in scat[NCHUNK - 1]:
        s.wait()

    plsc.subcore_barrier()
    pltpu.sync_copy(s2_sh.at[pl.ds(sid * RPT, RPT)],
                    s2p_h.at[cid, pl.ds(sid * RPT, RPT)])


def _k5_body(s2p_ref, sc_ref, srn_ref, rb_ref, out_ref):
    s2 = (s2p_ref[0] + s2p_ref[1])[:N, :C]
    r = rb_ref[0:1]
    bc = rb_ref[1:2]
    out_ref[...] = sc_ref[...] * s2 + srn_ref[...] * r + bc


def kernel(features, edge_index, edge_weight, node_id, alpha,
           W0, b0, W1, b1, Wout, bout):
    src = edge_index[0]
    dst = edge_index[1]
    ew = edge_weight[:, 0]
    pad = EPAD - E
    pad_ar = jnp.arange(pad, dtype=_i32)
    src_p = jnp.concatenate([src, pad_ar % N])
    dst_p = jnp.concatenate([dst, N + pad_ar % (NPAD - N)])
    ew_p = jnp.concatenate([ew, jnp.zeros((pad,), _f32)])
    dst2d = dst_p.reshape(EPAD // 128, 128)
    al_p = jnp.concatenate([alpha[:, 0], jnp.zeros((ALP - GENE - 2,), _f32)])

    zp, rb = pl.pallas_call(
        _k1_body,
        out_shape=[jax.ShapeDtypeStruct((NP, N), _i32),
                   jax.ShapeDtypeStruct((2, C), _f32)],
    )(features, W0, W1, Wout, b0[None, :], b1[None, :], bout[None, :])

    mesh = plsc.VectorSubcoreMesh(core_axis_name="c", subcore_axis_name="s")
    params = pltpu.CompilerParams(needs_layout_passes=False,
                                  use_tc_tiling_on_sc=False)

    pass1 = pl.kernel(
        _pass1_body,
        out_type=[jax.ShapeDtypeStruct((NC, NPAD, F1), _f32),
                  jax.ShapeDtypeStruct((EPAD,), _f32)],
        mesh=mesh,
        compiler_params=params,
        scratch_types=[pltpu.VMEM((N,), _i32),
                       pltpu.VMEM((ALP,), _f32),
                       pltpu.VMEM((NP * N,), _i32),
                       [pltpu.VMEM((CH,), _i32)] * 3,
                       [pltpu.VMEM((NSTR, 128), _i32)] * 3,
                       [pltpu.VMEM((CH,), _f32)] * 3,
                       [pltpu.VMEM((CH,), _f32)] * 3,
                       [pltpu.VMEM((CH, F1), _f32)] * 2,
                       pltpu.VMEM_SHARED((NPAD, F1), _f32),
                       [pltpu.SemaphoreType.DMA] * 3,
                       [pltpu.SemaphoreType.DMA] * 2,
                       [pltpu.SemaphoreType.DMA] * 3],
    )
    s1p, cedge = pass1(src_p, dst2d, ew_p, node_id, al_p, zp)

    azp, scl, srn = pl.pallas_call(
        _k3_body,
        out_shape=[jax.ShapeDtypeStruct((NP, N), _i32),
                   jax.ShapeDtypeStruct((N, 1), _f32),
                   jax.ShapeDtypeStruct((N, 1), _f32)],
    )(s1p)

    pass2 = pl.kernel(
        _pass2_body,
        out_type=[jax.ShapeDtypeStruct((NC, NPAD, F2), _f32)],
        mesh=mesh,
        compiler_params=params,
        scratch_types=[pltpu.VMEM((NP * N,), _i32),
                       [pltpu.VMEM((CH,), _i32)] * 3,
                       [pltpu.VMEM((NSTR, 128), _i32)] * 3,
                       [pltpu.VMEM((CH,), _f32)] * 3,
                       [pltpu.VMEM((CH, F2), _f32)] * 2,
                       [pltpu.SemaphoreType.DMA] * 3,
                       [pltpu.SemaphoreType.DMA] * 2,
                       pltpu.VMEM_SHARED((NPAD, F2), _f32)],
    )
    (s2p,) = pass2(src_p, dst2d, cedge, azp)

    out = pl.pallas_call(
        _k5_body,
        out_shape=jax.ShapeDtypeStruct((N, C), _f32),
    )(s2p, scl, srn, rb)
    return out


# speedup vs baseline: 59.9836x; 1.0557x over previous
import jax
import jax.numpy as jnp
from jax import lax
from jax.experimental import pallas as pl
from jax.experimental.pallas import tpu as pltpu
from jax.experimental.pallas import tpu_sc as plsc

N = 10000
NPAD = 10240
E = 320000
D = 128
C = 10
NP = 5
F1 = 16
F2 = 16
GENE = 5000
ALP = 5008
NC, NS, L = 2, 16, 16
NW = NC * NS
EPW = E // NW
CH = 1024
NCHUNK = 10
CBASE = [0, 1024, 2048, 3072, 4096, 5120, 6144, 7168, 8192, 8976]
TAIL_FROM = 240
NG = CH // L
NSTR = CH // 128
RPT = NPAD // NS

_f32 = jnp.float32
_i32 = jnp.int32


def _pack_bf16_pairs_t(xt):
    b = lax.bitcast_convert_type(xt, _i32)
    r = b + jnp.int32(0x7FFF) + (lax.shift_right_logical(b, 16) & 1)
    h = lax.shift_right_logical(r, 16)
    rows = [h[2 * j:2 * j + 1, :]
            | lax.shift_left(h[2 * j + 1:2 * j + 2, :], 16)
            for j in range(NP)]
    return jnp.concatenate(rows, axis=0)


def _k1_body(x_ref, w0_ref, w1_ref, wout_ref, b0_ref, b1_ref, bout_ref,
             zp_ref, rb_ref):
    wc = jnp.dot(wout_ref[...], w1_ref[...], preferred_element_type=_f32)
    wd = jnp.dot(wc, w0_ref[...], preferred_element_type=_f32)
    zt = lax.dot_general(wd, x_ref[...], (((1,), (1,)), ((), ())),
                         preferred_element_type=_f32)
    zp_ref[...] = _pack_bf16_pairs_t(zt)
    r = jnp.dot(b0_ref[...], wc.T, preferred_element_type=_f32)
    bc = jnp.dot(b1_ref[...], wout_ref[...].T,
                 preferred_element_type=_f32) + bout_ref[...]
    rb_ref[...] = jnp.concatenate([r, bc], axis=0)


def _zero_acc(zs, sh, sid, width):
    lane = lax.iota(_i32, L)
    zeros16 = jnp.zeros((L,), _f32)

    def _z(g, _):
        rowi = g * L + lane
        for j in range(width):
            plsc.store_scatter(zs, [rowi, jnp.full((L,), j, _i32)], zeros16)
        return 0
    lax.fori_loop(0, RPT // L, _z, 0)
    pltpu.sync_copy(zs.at[pl.ds(0, RPT)], sh.at[pl.ds(sid * RPT, RPT)])
    plsc.subcore_barrier()


def _unpack_cols(w):
    lo = plsc.bitcast(lax.shift_left(w, 16), _f32)
    hi = plsc.bitcast(w & jnp.int32(-65536), _f32)
    return lo, hi


def _pass1_body(ei_h, ew_h, nid_h, al_h, zp_h,
                s1p_h, c_h,
                nid_v, al_v, zp_v, src_v, dst2_v, ew_v, c_v, zs_v,
                s1_sh, lsem, ssem, wsem):
    cid = lax.axis_index("c")
    sid = lax.axis_index("s")
    w = cid * NS + sid
    lane = lax.iota(_i32, L)
    ones16 = jnp.ones((L,), _f32)

    pltpu.sync_copy(nid_h, nid_v)
    pltpu.sync_copy(al_h, al_v)
    for j in range(NP):
        pltpu.sync_copy(zp_h.at[j], zp_v.at[pl.ds(j * N, N)])
    _zero_acc(zs_v[0], s1_sh, sid, F1)

    def fire_loads(k):
        b = k % 3
        base = pl.multiple_of(w * EPW + CBASE[k], 16)
        ds = [pltpu.async_copy(ei_h.at[0, pl.ds(base, CH)], src_v[b], lsem[b]),
              pltpu.async_copy(ew_h.at[pl.ds(base, CH)], ew_v[b], lsem[b])]
        for j in range(NSTR):
            ds.append(pltpu.async_copy(
                ei_h.at[1, pl.ds(pl.multiple_of(base + j * 128, 16), 128)],
                dst2_v[b].at[j], lsem[b]))
        return ds

    def compute(k):
        b, p = k % 3, k % 2
        tail = k == NCHUNK - 1

        def _g(g, _):
            s16 = src_v[b][pl.ds(g * L, L)]
            d16 = dst2_v[b][g // 8, pl.ds((g % 8) * L, L)]
            sid16 = plsc.load_gather(nid_v, [s16])
            did16 = plsc.load_gather(nid_v, [d16])
            sg = sid16 >= 0
            dg = did16 >= 0
            idx16 = jnp.full((L,), GENE + 1, _i32)
            idx16 = jnp.where(sg & (~dg), sid16, idx16)
            idx16 = jnp.where(dg & (~sg), did16, idx16)
            idx16 = jnp.where(dg & sg, jnp.full((L,), GENE, _i32), idx16)
            a16 = plsc.load_gather(al_v, [idx16])
            c16 = a16 * ew_v[b][pl.ds(g * L, L)]
            rowi = g * L + lane
            valid = jnp.ones((L,), _f32) if not tail else jnp.where(
                rowi >= TAIL_FROM, 1.0, 0.0)
            if tail:
                c16 = c16 * valid
            c_v[b][pl.ds(g * L, L)] = c16
            for jp in range(NP):
                wrd = plsc.load_gather(zp_v, [s16 + (jp * N)])
                lo, hi = _unpack_cols(wrd)
                plsc.store_scatter(zs_v[p], [rowi, jnp.full((L,), 2 * jp,
                                                            _i32)], lo * c16)
                plsc.store_scatter(zs_v[p], [rowi, jnp.full((L,), 2 * jp + 1,
                                                            _i32)], hi * c16)
            plsc.store_scatter(zs_v[p], [rowi, jnp.full((L,), 14, _i32)],
                               valid)
            plsc.store_scatter(zs_v[p], [rowi, jnp.full((L,), 15, _i32)], c16)
            return 0
        lax.fori_loop(0, NG, _g, 0)
        base = pl.multiple_of(w * EPW + CBASE[k], 16)
        if not tail:
            return pltpu.async_copy(c_v[b], c_h.at[pl.ds(base, CH)], wsem[b])
        return pltpu.async_copy(
            c_v[b].at[pl.ds(TAIL_FROM, CH - TAIL_FROM)],
            c_h.at[pl.ds(pl.multiple_of(base + TAIL_FROM, 16),
                         CH - TAIL_FROM)], wsem[b])

    def fire_scatters(k):
        b, p = k % 3, k % 2
        return [pltpu.async_copy(zs_v[p].at[pl.ds(j * 128, 128)],
                                 s1_sh.at[dst2_v[b].at[j]], ssem[p], add=True)
                for j in range(NSTR)]

    loads = {0: fire_loads(0)}
    scat, cw = {}, {}
    for k in range(NCHUNK):
        if k >= 2:
            for s in scat[k - 2]:
                s.wait()
        if k >= 3:
            cw[k - 3].wait()
        for d in loads[k]:
            d.wait()
        if k + 1 < NCHUNK:
            loads[k + 1] = fire_loads(k + 1)
        cw[k] = compute(k)
        scat[k] = fire_scatters(k)
    for s in scat[NCHUNK - 2]:
        s.wait()
    for s in scat[NCHUNK - 1]:
        s.wait()
    for k in range(NCHUNK - 3, NCHUNK):
        cw[k].wait()

    plsc.subcore_barrier()
    pltpu.sync_copy(s1_sh.at[pl.ds(sid * RPT, RPT)],
                    s1p_h.at[cid, pl.ds(sid * RPT, RPT)])


def _k3_body(s1p_ref, azp_ref, sc_ref, srn_ref):
    s = s1p_ref[0] + s1p_ref[1]
    cnt = s[:, 14:15]
    rs = s[:, 15:16]
    scale = 1.0 / jnp.maximum(cnt, 1.0)
    az = (s[:, :C] * scale)[:N]
    azp_ref[...] = _pack_bf16_pairs_t(az.T)
    sc_ref[...] = scale[:N]
    srn_ref[...] = (scale * rs)[:N]


def _pass2_body(ei_h, cin_h, azp_h,
                s2p_h,
                azp_v, src_v, dst2_v, c_v, zs_v, lsem, ssem, s2_sh):
    cid = lax.axis_index("c")
    sid = lax.axis_index("s")
    w = cid * NS + sid
    lane = lax.iota(_i32, L)

    for j in range(NP):
        pltpu.sync_copy(azp_h.at[j], azp_v.at[pl.ds(j * N, N)])
    _zero_acc(zs_v[0], s2_sh, sid, F2)

    def fire_loads(k):
        b = k % 3
        base = pl.multiple_of(w * EPW + CBASE[k], 16)
        ds = [pltpu.async_copy(ei_h.at[0, pl.ds(base, CH)], src_v[b], lsem[b]),
              pltpu.async_copy(cin_h.at[pl.ds(base, CH)], c_v[b], lsem[b])]
        for j in range(NSTR):
            ds.append(pltpu.async_copy(
                ei_h.at[1, pl.ds(pl.multiple_of(base + j * 128, 16), 128)],
                dst2_v[b].at[j], lsem[b]))
        return ds

    def compute(k):
        b, p = k % 3, k % 2
        tail = k == NCHUNK - 1

        def _g(g, _):
            s16 = src_v[b][pl.ds(g * L, L)]
            c16 = c_v[b][pl.ds(g * L, L)]
            rowi = g * L + lane
            if tail:
                c16 = jnp.where(rowi >= TAIL_FROM, c16, 0.0)
            for jp in range(NP):
                wrd = plsc.load_gather(azp_v, [s16 + (jp * N)])
                lo, hi = _unpack_cols(wrd)
                plsc.store_scatter(zs_v[p], [rowi, jnp.full((L,), 2 * jp,
                                                            _i32)], lo * c16)
                plsc.store_scatter(zs_v[p], [rowi, jnp.full((L,), 2 * jp + 1,
                                                            _i32)], hi * c16)
            return 0
        lax.fori_loop(0, NG, _g, 0)

    def fire_scatters(k):
        b, p = k % 3, k % 2
        return [pltpu.async_copy(zs_v[p].at[pl.ds(j * 128, 128)],
                                 s2_sh.at[dst2_v[b].at[j]], ssem[p], add=True)
                for j in range(NSTR)]

    loads = {0: fire_loads(0)}
    scat = {}
    for k in range(NCHUNK):
        if k >= 2:
            for s in scat[k - 2]:
                s.wait()
        for d in loads[k]:
            d.wait()
        if k + 1 < NCHUNK:
            loads[k + 1] = fire_loads(k + 1)
        compute(k)
        scat[k] = fire_scatters(k)
    for s in scat[NCHUNK - 2]:
        s.wait()
    for s in scat[NCHUNK - 1]:
        s.wait()

    plsc.subcore_barrier()
    pltpu.sync_copy(s2_sh.at[pl.ds(sid * RPT, RPT)],
                    s2p_h.at[cid, pl.ds(sid * RPT, RPT)])


def _k5_body(s2p_ref, sc_ref, srn_ref, rb_ref, out_ref):
    s2 = (s2p_ref[0] + s2p_ref[1])[:N, :C]
    r = rb_ref[0:1]
    bc = rb_ref[1:2]
    out_ref[...] = sc_ref[...] * s2 + srn_ref[...] * r + bc


def kernel(features, edge_index, edge_weight, node_id, alpha,
           W0, b0, W1, b1, Wout, bout):
    ew_f = edge_weight.reshape(E)
    al_p = jnp.concatenate([alpha[:, 0], jnp.zeros((ALP - GENE - 2,), _f32)])

    zp, rb = pl.pallas_call(
        _k1_body,
        out_shape=[jax.ShapeDtypeStruct((NP, N), _i32),
                   jax.ShapeDtypeStruct((2, C), _f32)],
    )(features, W0, W1, Wout, b0[None, :], b1[None, :], bout[None, :])

    mesh = plsc.VectorSubcoreMesh(core_axis_name="c", subcore_axis_name="s")
    params = pltpu.CompilerParams(needs_layout_passes=False,
                                  use_tc_tiling_on_sc=False)

    pass1 = pl.kernel(
        _pass1_body,
        out_type=[jax.ShapeDtypeStruct((NC, NPAD, F1), _f32),
                  jax.ShapeDtypeStruct((E,), _f32)],
        mesh=mesh,
        compiler_params=params,
        scratch_types=[pltpu.VMEM((N,), _i32),
                       pltpu.VMEM((ALP,), _f32),
                       pltpu.VMEM((NP * N,), _i32),
                       [pltpu.VMEM((CH,), _i32)] * 3,
                       [pltpu.VMEM((NSTR, 128), _i32)] * 3,
                       [pltpu.VMEM((CH,), _f32)] * 3,
                       [pltpu.VMEM((CH,), _f32)] * 3,
                       [pltpu.VMEM((CH, F1), _f32)] * 2,
                       pltpu.VMEM_SHARED((NPAD, F1), _f32),
                       [pltpu.SemaphoreType.DMA] * 3,
                       [pltpu.SemaphoreType.DMA] * 2,
                       [pltpu.SemaphoreType.DMA] * 3],
    )
    s1p, cedge = pass1(edge_index, ew_f, node_id, al_p, zp)

    azp, scl, srn = pl.pallas_call(
        _k3_body,
        out_shape=[jax.ShapeDtypeStruct((NP, N), _i32),
                   jax.ShapeDtypeStruct((N, 1), _f32),
                   jax.ShapeDtypeStruct((N, 1), _f32)],
    )(s1p)

    pass2 = pl.kernel(
        _pass2_body,
        out_type=[jax.ShapeDtypeStruct((NC, NPAD, F2), _f32)],
        mesh=mesh,
        compiler_params=params,
        scratch_types=[pltpu.VMEM((NP * N,), _i32),
                       [pltpu.VMEM((CH,), _i32)] * 3,
                       [pltpu.VMEM((NSTR, 128), _i32)] * 3,
                       [pltpu.VMEM((CH,), _f32)] * 3,
                       [pltpu.VMEM((CH, F2), _f32)] * 2,
                       [pltpu.SemaphoreType.DMA] * 3,
                       [pltpu.SemaphoreType.DMA] * 2,
                       pltpu.VMEM_SHARED((NPAD, F2), _f32)],
    )
    (s2p,) = pass2(edge_index, cedge, azp)

    out = pl.pallas_call(
        _k5_body,
        out_shape=jax.ShapeDtypeStruct((N, C), _f32),
    )(s2p, scl, srn, rb)
    return out


# speedup vs baseline: 65.7935x; 1.0969x over previous
import jax
import jax.numpy as jnp
from jax import lax
from jax.experimental import pallas as pl
from jax.experimental.pallas import tpu as pltpu
from jax.experimental.pallas import tpu_sc as plsc

N = 10000
NPAD = 10240
E = 320000
D = 128
C = 10
NP = 5
F1 = 16
F2 = 16
GENE = 5000
ALP = 5008
NC, NS, L = 2, 16, 16
NW = NC * NS
EPW = E // NW
CH = 1024
NCHUNK = 10
CBASE = [0, 1024, 2048, 3072, 4096, 5120, 6144, 7168, 8192, 8976]
TAIL_FROM = 240
NG = CH // L
NSTR = CH // 128
RPT = NPAD // NS
RW = NPAD // NW
RWL = N - (NW - 1) * RW

_f32 = jnp.float32
_i32 = jnp.int32


def _pack_bf16_pairs_t(xt):
    b = lax.bitcast_convert_type(xt, _i32)
    r = b + jnp.int32(0x7FFF) + (lax.shift_right_logical(b, 16) & 1)
    h = lax.shift_right_logical(r, 16)
    rows = [h[2 * j:2 * j + 1, :]
            | lax.shift_left(h[2 * j + 1:2 * j + 2, :], 16)
            for j in range(NP)]
    return jnp.concatenate(rows, axis=0)


def _k1_body(x_ref, w0_ref, w1_ref, wout_ref, b0_ref, b1_ref, bout_ref,
             zp_ref, rb_ref):
    wc = jnp.dot(wout_ref[...], w1_ref[...], preferred_element_type=_f32)
    wd = jnp.dot(wc, w0_ref[...], preferred_element_type=_f32)
    zt = lax.dot_general(wd, x_ref[...], (((1,), (1,)), ((), ())),
                         preferred_element_type=_f32)
    zp_ref[...] = _pack_bf16_pairs_t(zt)
    r = jnp.dot(b0_ref[...], wc.T, preferred_element_type=_f32)
    bc = jnp.dot(b1_ref[...], wout_ref[...].T,
                 preferred_element_type=_f32) + bout_ref[...]
    rb_ref[...] = jnp.concatenate([r, bc], axis=0)


def _zero_acc(zs, sh, sid, width):
    lane = lax.iota(_i32, L)
    zeros16 = jnp.zeros((L,), _f32)

    def _z(g, _):
        rowi = g * L + lane
        for j in range(width):
            plsc.store_scatter(zs, [rowi, jnp.full((L,), j, _i32)], zeros16)
        return 0
    lax.fori_loop(0, RPT // L, _z, 0)
    pltpu.sync_copy(zs.at[pl.ds(0, RPT)], sh.at[pl.ds(sid * RPT, RPT)])
    plsc.subcore_barrier()


def _unpack_cols(w):
    lo = plsc.bitcast(lax.shift_left(w, 16), _f32)
    hi = plsc.bitcast(w & jnp.int32(-65536), _f32)
    return lo, hi


def _bf16_round_bits(x):
    b = plsc.bitcast(x, _i32)
    return b + jnp.int32(0x7FFF) + (lax.shift_right_logical(b, 16) & 1)


def _pass1_body(ei_h, ew_h, nid_h, al_h, zp_h,
                s1p_h, c_h,
                nid_v, al_v, zp_v, src_v, dst2_v, ew_v, c_v, zs_v,
                s1_sh, lsem, ssem, wsem):
    cid = lax.axis_index("c")
    sid = lax.axis_index("s")
    w = cid * NS + sid
    lane = lax.iota(_i32, L)

    pltpu.sync_copy(nid_h, nid_v)
    pltpu.sync_copy(al_h, al_v)
    for j in range(NP):
        pltpu.sync_copy(zp_h.at[j], zp_v.at[pl.ds(j * N, N)])
    _zero_acc(zs_v[0], s1_sh, sid, F1)

    def fire_loads(k):
        b = k % 3
        base = pl.multiple_of(w * EPW + CBASE[k], 16)
        ds = [pltpu.async_copy(ei_h.at[0, pl.ds(base, CH)], src_v[b], lsem[b]),
              pltpu.async_copy(ew_h.at[pl.ds(base, CH)], ew_v[b], lsem[b])]
        for j in range(NSTR):
            ds.append(pltpu.async_copy(
                ei_h.at[1, pl.ds(pl.multiple_of(base + j * 128, 16), 128)],
                dst2_v[b].at[j], lsem[b]))
        return ds

    def compute(k):
        b, p = k % 3, k % 2
        tail = k == NCHUNK - 1

        def _g(g, _):
            s16 = src_v[b][pl.ds(g * L, L)]
            d16 = dst2_v[b][g // 8, pl.ds((g % 8) * L, L)]
            sid16 = plsc.load_gather(nid_v, [s16])
            did16 = plsc.load_gather(nid_v, [d16])
            sg = sid16 >= 0
            dg = did16 >= 0
            idx16 = jnp.full((L,), GENE + 1, _i32)
            idx16 = jnp.where(sg & (~dg), sid16, idx16)
            idx16 = jnp.where(dg & (~sg), did16, idx16)
            idx16 = jnp.where(dg & sg, jnp.full((L,), GENE, _i32), idx16)
            a16 = plsc.load_gather(al_v, [idx16])
            c16 = a16 * ew_v[b][pl.ds(g * L, L)]
            rowi = g * L + lane
            valid = jnp.ones((L,), _f32) if not tail else jnp.where(
                rowi >= TAIL_FROM, 1.0, 0.0)
            if tail:
                c16 = c16 * valid
            c_v[b][pl.ds(g * L, L)] = c16
            for jp in range(NP):
                wrd = plsc.load_gather(zp_v, [s16 + (jp * N)])
                lo, hi = _unpack_cols(wrd)
                plsc.store_scatter(zs_v[p], [rowi, jnp.full((L,), 2 * jp,
                                                            _i32)], lo * c16)
                plsc.store_scatter(zs_v[p], [rowi, jnp.full((L,), 2 * jp + 1,
                                                            _i32)], hi * c16)
            plsc.store_scatter(zs_v[p], [rowi, jnp.full((L,), 14, _i32)],
                               valid)
            plsc.store_scatter(zs_v[p], [rowi, jnp.full((L,), 15, _i32)], c16)
            return 0
        lax.fori_loop(0, NG, _g, 0)
        base = pl.multiple_of(w * EPW + CBASE[k], 16)
        if not tail:
            return pltpu.async_copy(c_v[b], c_h.at[pl.ds(base, CH)], wsem[b])
        return pltpu.async_copy(
            c_v[b].at[pl.ds(TAIL_FROM, CH - TAIL_FROM)],
            c_h.at[pl.ds(pl.multiple_of(base + TAIL_FROM, 16),
                         CH - TAIL_FROM)], wsem[b])

    def fire_scatters(k):
        b, p = k % 3, k % 2
        return [pltpu.async_copy(zs_v[p].at[pl.ds(j * 128, 128)],
                                 s1_sh.at[dst2_v[b].at[j]], ssem[p], add=True)
                for j in range(NSTR)]

    loads = {0: fire_loads(0)}
    scat, cw = {}, {}
    for k in range(NCHUNK):
        if k >= 2:
            for s in scat[k - 2]:
                s.wait()
        if k >= 3:
            cw[k - 3].wait()
        for d in loads[k]:
            d.wait()
        if k + 1 < NCHUNK:
            loads[k + 1] = fire_loads(k + 1)
        cw[k] = compute(k)
        scat[k] = fire_scatters(k)
    for s in scat[NCHUNK - 2]:
        s.wait()
    for s in scat[NCHUNK - 1]:
        s.wait()
    for k in range(NCHUNK - 3, NCHUNK):
        cw[k].wait()

    plsc.subcore_barrier()
    pltpu.sync_copy(s1_sh.at[pl.ds(sid * RPT, RPT)],
                    s1p_h.at[cid, pl.ds(sid * RPT, RPT)])


def _k3_body(s1p_h, azp_h, scl_h, srn_h, a_v, b_v, azp_v, scl_v, srn_v):
    cid = lax.axis_index("c")
    sid = lax.axis_index("s")
    w = cid * NS + sid
    last = w == NW - 1
    base = pl.multiple_of(w * RW, 16)
    lane = lax.iota(_i32, L)

    pltpu.sync_copy(s1p_h.at[0, pl.ds(base, RW)], a_v)
    pltpu.sync_copy(s1p_h.at[1, pl.ds(base, RW)], b_v)

    def _gg(ref, col, rowi):
        return plsc.load_gather(ref, [rowi, jnp.full((L,), col, _i32)])

    def _grp(gi, _):
        rowi = gi * L + lane
        cnt = _gg(a_v, 14, rowi) + _gg(b_v, 14, rowi)
        scale = 1.0 / jnp.maximum(cnt, 1.0)
        rs = _gg(a_v, 15, rowi) + _gg(b_v, 15, rowi)
        scl_v[pl.ds(gi * L, L)] = scale
        srn_v[pl.ds(gi * L, L)] = scale * rs
        for jp in range(NP):
            ev = (_gg(a_v, 2 * jp, rowi) + _gg(b_v, 2 * jp, rowi)) * scale
            od = (_gg(a_v, 2 * jp + 1, rowi) + _gg(b_v, 2 * jp + 1,
                                                   rowi)) * scale
            word = (lax.shift_right_logical(_bf16_round_bits(ev), 16)
                    | (_bf16_round_bits(od) & jnp.int32(-65536)))
            plsc.store_scatter(azp_v, [jnp.full((L,), jp, _i32), rowi], word)
        return 0
    lax.fori_loop(0, RW // L, _grp, 0)

    @pl.when(jnp.logical_not(last))
    def _full():
        for j in range(NP):
            pltpu.sync_copy(azp_v.at[j], azp_h.at[j, pl.ds(base, RW)])
        pltpu.sync_copy(scl_v, scl_h.at[pl.ds(base, RW)])
        pltpu.sync_copy(srn_v, srn_h.at[pl.ds(base, RW)])

    @pl.when(last)
    def _part():
        for j in range(NP):
            pltpu.sync_copy(azp_v.at[j, pl.ds(0, RWL)],
                            azp_h.at[j, pl.ds(base, RWL)])
        pltpu.sync_copy(scl_v.at[pl.ds(0, RWL)], scl_h.at[pl.ds(base, RWL)])
        pltpu.sync_copy(srn_v.at[pl.ds(0, RWL)], srn_h.at[pl.ds(base, RWL)])


def _pass2_body(ei_h, cin_h, azp_h,
                s2p_h,
                azp_v, src_v, dst2_v, c_v, zs_v, lsem, ssem, s2_sh):
    cid = lax.axis_index("c")
    sid = lax.axis_index("s")
    w = cid * NS + sid
    lane = lax.iota(_i32, L)

    for j in range(NP):
        pltpu.sync_copy(azp_h.at[j], azp_v.at[pl.ds(j * N, N)])
    _zero_acc(zs_v[0], s2_sh, sid, F2)

    def fire_loads(k):
        b = k % 3
        base = pl.multiple_of(w * EPW + CBASE[k], 16)
        ds = [pltpu.async_copy(ei_h.at[0, pl.ds(base, CH)], src_v[b], lsem[b]),
              pltpu.async_copy(cin_h.at[pl.ds(base, CH)], c_v[b], lsem[b])]
        for j in range(NSTR):
            ds.append(pltpu.async_copy(
                ei_h.at[1, pl.ds(pl.multiple_of(base + j * 128, 16), 128)],
                dst2_v[b].at[j], lsem[b]))
        return ds

    def compute(k):
        b, p = k % 3, k % 2
        tail = k == NCHUNK - 1

        def _g(g, _):
            s16 = src_v[b][pl.ds(g * L, L)]
            c16 = c_v[b][pl.ds(g * L, L)]
            rowi = g * L + lane
            if tail:
                c16 = jnp.where(rowi >= TAIL_FROM, c16, 0.0)
            for jp in range(NP):
                wrd = plsc.load_gather(azp_v, [s16 + (jp * N)])
                lo, hi = _unpack_cols(wrd)
                plsc.store_scatter(zs_v[p], [rowi, jnp.full((L,), 2 * jp,
                                                            _i32)], lo * c16)
                plsc.store_scatter(zs_v[p], [rowi, jnp.full((L,), 2 * jp + 1,
                                                            _i32)], hi * c16)
            return 0
        lax.fori_loop(0, NG, _g, 0)

    def fire_scatters(k):
        b, p = k % 3, k % 2
        return [pltpu.async_copy(zs_v[p].at[pl.ds(j * 128, 128)],
                                 s2_sh.at[dst2_v[b].at[j]], ssem[p], add=True)
                for j in range(NSTR)]

    loads = {0: fire_loads(0)}
    scat = {}
    for k in range(NCHUNK):
        if k >= 2:
            for s in scat[k - 2]:
                s.wait()
        for d in loads[k]:
            d.wait()
        if k + 1 < NCHUNK:
            loads[k + 1] = fire_loads(k + 1)
        compute(k)
        scat[k] = fire_scatters(k)
    for s in scat[NCHUNK - 2]:
        s.wait()
    for s in scat[NCHUNK - 1]:
        s.wait()

    plsc.subcore_barrier()
    pltpu.sync_copy(s2_sh.at[pl.ds(sid * RPT, RPT)],
                    s2p_h.at[cid, pl.ds(sid * RPT, RPT)])


def _k5_body(s2p_h, scl_h, srn_h, rb_h, out_h,
             a_v, b_v, scl_v, srn_v, rb_v, out_v):
    cid = lax.axis_index("c")
    sid = lax.axis_index("s")
    w = cid * NS + sid
    last = w == NW - 1
    base = pl.multiple_of(w * RW, 16)
    lane = lax.iota(_i32, L)

    pltpu.sync_copy(s2p_h.at[0, pl.ds(base, RW)], a_v)
    pltpu.sync_copy(s2p_h.at[1, pl.ds(base, RW)], b_v)
    pltpu.sync_copy(rb_h, rb_v)

    @pl.when(jnp.logical_not(last))
    def _ldf():
        pltpu.sync_copy(scl_h.at[pl.ds(base, RW)], scl_v)
        pltpu.sync_copy(srn_h.at[pl.ds(base, RW)], srn_v)

    @pl.when(last)
    def _ldp():
        pltpu.sync_copy(scl_h.at[pl.ds(base, RWL)], scl_v.at[pl.ds(0, RWL)])
        pltpu.sync_copy(srn_h.at[pl.ds(base, RWL)], srn_v.at[pl.ds(0, RWL)])

    zero16 = jnp.zeros((L,), _i32)
    rjs = [plsc.load_gather(rb_v, [zero16, jnp.full((L,), j, _i32)])
           for j in range(C)]
    bcs = [plsc.load_gather(rb_v, [jnp.full((L,), 1, _i32),
                                   jnp.full((L,), j, _i32)])
           for j in range(C)]

    def _gg(ref, col, rowi):
        return plsc.load_gather(ref, [rowi, jnp.full((L,), col, _i32)])

    def _grp(gi, _):
        rowi = gi * L + lane
        scale = scl_v[pl.ds(gi * L, L)]
        srn = srn_v[pl.ds(gi * L, L)]
        for j in range(C):
            col = ((_gg(a_v, j, rowi) + _gg(b_v, j, rowi)) * scale
                   + srn * rjs[j] + bcs[j])
            plsc.store_scatter(out_v, [rowi, jnp.full((L,), j, _i32)], col)
        return 0
    lax.fori_loop(0, RW // L, _grp, 0)

    @pl.when(jnp.logical_not(last))
    def _stf():
        pltpu.sync_copy(out_v, out_h.at[pl.ds(base, RW)])

    @pl.when(last)
    def _stp():
        pltpu.sync_copy(out_v.at[pl.ds(0, RWL)], out_h.at[pl.ds(base, RWL)])


def kernel(features, edge_index, edge_weight, node_id, alpha,
           W0, b0, W1, b1, Wout, bout):
    ew_f = edge_weight.reshape(E)
    al_p = jnp.concatenate([alpha[:, 0], jnp.zeros((ALP - GENE - 2,), _f32)])

    zp, rb = pl.pallas_call(
        _k1_body,
        out_shape=[jax.ShapeDtypeStruct((NP, N), _i32),
                   jax.ShapeDtypeStruct((2, C), _f32)],
    )(features, W0, W1, Wout, b0[None, :], b1[None, :], bout[None, :])

    mesh = plsc.VectorSubcoreMesh(core_axis_name="c", subcore_axis_name="s")
    params = pltpu.CompilerParams(needs_layout_passes=False,
                                  use_tc_tiling_on_sc=False)

    pass1 = pl.kernel(
        _pass1_body,
        out_type=[jax.ShapeDtypeStruct((NC, NPAD, F1), _f32),
                  jax.ShapeDtypeStruct((E,), _f32)],
        mesh=mesh,
        compiler_params=params,
        scratch_types=[pltpu.VMEM((N,), _i32),
                       pltpu.VMEM((ALP,), _f32),
                       pltpu.VMEM((NP * N,), _i32),
                       [pltpu.VMEM((CH,), _i32)] * 3,
                       [pltpu.VMEM((NSTR, 128), _i32)] * 3,
                       [pltpu.VMEM((CH,), _f32)] * 3,
                       [pltpu.VMEM((CH,), _f32)] * 3,
                       [pltpu.VMEM((CH, F1), _f32)] * 2,
                       pltpu.VMEM_SHARED((NPAD, F1), _f32),
                       [pltpu.SemaphoreType.DMA] * 3,
                       [pltpu.SemaphoreType.DMA] * 2,
                       [pltpu.SemaphoreType.DMA] * 3],
    )
    s1p, cedge = pass1(edge_index, ew_f, node_id, al_p, zp)

    k3 = pl.kernel(
        _k3_body,
        out_type=[jax.ShapeDtypeStruct((NP, N), _i32),
                  jax.ShapeDtypeStruct((N,), _f32),
                  jax.ShapeDtypeStruct((N,), _f32)],
        mesh=mesh,
        compiler_params=params,
        scratch_types=[pltpu.VMEM((RW, F1), _f32),
                       pltpu.VMEM((RW, F1), _f32),
                       pltpu.VMEM((NP, RW), _i32),
                       pltpu.VMEM((RW,), _f32),
                       pltpu.VMEM((RW,), _f32)],
    )
    azp, scl, srn = k3(s1p)

    pass2 = pl.kernel(
        _pass2_body,
        out_type=[jax.ShapeDtypeStruct((NC, NPAD, F2), _f32)],
        mesh=mesh,
        compiler_params=params,
        scratch_types=[pltpu.VMEM((NP * N,), _i32),
                       [pltpu.VMEM((CH,), _i32)] * 3,
                       [pltpu.VMEM((NSTR, 128), _i32)] * 3,
                       [pltpu.VMEM((CH,), _f32)] * 3,
                       [pltpu.VMEM((CH, F2), _f32)] * 2,
                       [pltpu.SemaphoreType.DMA] * 3,
                       [pltpu.SemaphoreType.DMA] * 2,
                       pltpu.VMEM_SHARED((NPAD, F2), _f32)],
    )
    (s2p,) = pass2(edge_index, cedge, azp)

    k5 = pl.kernel(
        _k5_body,
        out_type=[jax.ShapeDtypeStruct((N, C), _f32)],
        mesh=mesh,
        compiler_params=params,
        scratch_types=[pltpu.VMEM((RW, F1), _f32),
                       pltpu.VMEM((RW, F1), _f32),
                       pltpu.VMEM((RW,), _f32),
                       pltpu.VMEM((RW,), _f32),
                       pltpu.VMEM((2, C), _f32),
                       pltpu.VMEM((RW, C), _f32)],
    )
    (out,) = k5(s2p, scl, srn, rb)
    return out


# speedup vs baseline: 69.5388x; 1.0569x over previous
import jax
import jax.numpy as jnp
from jax import lax
from jax.experimental import pallas as pl
from jax.experimental.pallas import tpu as pltpu
from jax.experimental.pallas import tpu_sc as plsc

N = 10000
NPAD = 10240
E = 320000
D = 128
C = 10
NP = 5
F1 = 16
F2 = 16
GENE = 5000
ALP = 5008
NC, NS, L = 2, 16, 16
NW = NC * NS
EPW = E // NW
CH = 1024
NCHUNK = 10
CBASE = [0, 1024, 2048, 3072, 4096, 5120, 6144, 7168, 8192, 8976]
TAIL_FROM = 240
NG = CH // L
NSTR = CH // 128
RPT = NPAD // NS
RW = NPAD // NW
RWL = N - (NW - 1) * RW

_f32 = jnp.float32
_i32 = jnp.int32


def _pack_bf16_pairs_t(xt):
    b = lax.bitcast_convert_type(xt, _i32)
    r = b + jnp.int32(0x7FFF) + (lax.shift_right_logical(b, 16) & 1)
    h = lax.shift_right_logical(r, 16)
    rows = [h[2 * j:2 * j + 1, :]
            | lax.shift_left(h[2 * j + 1:2 * j + 2, :], 16)
            for j in range(NP)]
    return jnp.concatenate(rows, axis=0)


def _k1_body(x_ref, w0_ref, w1_ref, wout_ref, b0_ref, b1_ref, bout_ref,
             zp_ref, rb_ref):
    wc = jnp.dot(wout_ref[...], w1_ref[...], preferred_element_type=_f32)
    wd = jnp.dot(wc, w0_ref[...], preferred_element_type=_f32)
    zt = lax.dot_general(wd, x_ref[...], (((1,), (1,)), ((), ())),
                         preferred_element_type=_f32)
    zp_ref[...] = _pack_bf16_pairs_t(zt)
    r = jnp.dot(b0_ref[...], wc.T, preferred_element_type=_f32)
    bc = jnp.dot(b1_ref[...], wout_ref[...].T,
                 preferred_element_type=_f32) + bout_ref[...]
    rb_ref[...] = jnp.concatenate([r, bc], axis=0)


def _zero_acc(zs, sh, sid, width):
    lane = lax.iota(_i32, L)
    zeros16 = jnp.zeros((L,), _f32)

    def _z(g, _):
        rowi = g * L + lane
        for j in range(width):
            plsc.store_scatter(zs, [rowi, jnp.full((L,), j, _i32)], zeros16)
        return 0
    lax.fori_loop(0, RPT // L, _z, 0)
    pltpu.sync_copy(zs.at[pl.ds(0, RPT)], sh.at[pl.ds(sid * RPT, RPT)])
    plsc.subcore_barrier()


def _unpack_cols(w):
    lo = plsc.bitcast(lax.shift_left(w, 16), _f32)
    hi = plsc.bitcast(w & jnp.int32(-65536), _f32)
    return lo, hi


def _bf16_round_bits(x):
    b = plsc.bitcast(x, _i32)
    return b + jnp.int32(0x7FFF) + (lax.shift_right_logical(b, 16) & 1)


def _pass1_body(ei_h, ew_h, nid_h, al_h, zp_h,
                s1p_h, c_h,
                nid_v, al_v, zp_v, src_v, dst2_v, ew_v, c_v, zs_v,
                s1_sh, lsem, ssem, wsem):
    cid = lax.axis_index("c")
    sid = lax.axis_index("s")
    w = cid * NS + sid
    lane = lax.iota(_i32, L)

    stage = [pltpu.async_copy(nid_h, nid_v, wsem[0]),
             pltpu.async_copy(al_h, al_v, wsem[0])]
    stage += [pltpu.async_copy(zp_h.at[j], zp_v.at[pl.ds(j * N, N)], wsem[0])
              for j in range(NP)]
    _zero_acc(zs_v[0], s1_sh, sid, F1)
    for d in stage:
        d.wait()

    def fire_loads(k):
        b = k % 3
        base = pl.multiple_of(w * EPW + CBASE[k], 16)
        ds = [pltpu.async_copy(ei_h.at[0, pl.ds(base, CH)], src_v[b], lsem[b]),
              pltpu.async_copy(ew_h.at[pl.ds(base, CH)], ew_v[b], lsem[b])]
        for j in range(NSTR):
            ds.append(pltpu.async_copy(
                ei_h.at[1, pl.ds(pl.multiple_of(base + j * 128, 16), 128)],
                dst2_v[b].at[j], lsem[b]))
        return ds

    def compute(k):
        b, p = k % 3, k % 2
        tail = k == NCHUNK - 1

        def _g(g, _):
            s16 = src_v[b][pl.ds(g * L, L)]
            d16 = dst2_v[b][g // 8, pl.ds((g % 8) * L, L)]
            sid16 = plsc.load_gather(nid_v, [s16])
            did16 = plsc.load_gather(nid_v, [d16])
            sg = sid16 >= 0
            dg = did16 >= 0
            idx16 = jnp.full((L,), GENE + 1, _i32)
            idx16 = jnp.where(sg & (~dg), sid16, idx16)
            idx16 = jnp.where(dg & (~sg), did16, idx16)
            idx16 = jnp.where(dg & sg, jnp.full((L,), GENE, _i32), idx16)
            a16 = plsc.load_gather(al_v, [idx16])
            c16 = a16 * ew_v[b][pl.ds(g * L, L)]
            rowi = g * L + lane
            valid = jnp.ones((L,), _f32) if not tail else jnp.where(
                rowi >= TAIL_FROM, 1.0, 0.0)
            if tail:
                c16 = c16 * valid
            c_v[b][pl.ds(g * L, L)] = c16
            for jp in range(NP):
                wrd = plsc.load_gather(zp_v, [s16 + (jp * N)])
                lo, hi = _unpack_cols(wrd)
                plsc.store_scatter(zs_v[p], [rowi, jnp.full((L,), 2 * jp,
                                                            _i32)], lo * c16)
                plsc.store_scatter(zs_v[p], [rowi, jnp.full((L,), 2 * jp + 1,
                                                            _i32)], hi * c16)
            plsc.store_scatter(zs_v[p], [rowi, jnp.full((L,), 14, _i32)],
                               valid)
            plsc.store_scatter(zs_v[p], [rowi, jnp.full((L,), 15, _i32)], c16)
            return 0
        lax.fori_loop(0, NG, _g, 0)
        base = pl.multiple_of(w * EPW + CBASE[k], 16)
        if not tail:
            return pltpu.async_copy(c_v[b], c_h.at[pl.ds(base, CH)], wsem[b])
        return pltpu.async_copy(
            c_v[b].at[pl.ds(TAIL_FROM, CH - TAIL_FROM)],
            c_h.at[pl.ds(pl.multiple_of(base + TAIL_FROM, 16),
                         CH - TAIL_FROM)], wsem[b])

    def fire_scatters(k):
        b, p = k % 3, k % 2
        return [pltpu.async_copy(zs_v[p].at[pl.ds(j * 128, 128)],
                                 s1_sh.at[dst2_v[b].at[j]], ssem[p], add=True)
                for j in range(NSTR)]

    loads = {0: fire_loads(0)}
    scat, cw = {}, {}
    for k in range(NCHUNK):
        if k >= 2:
            for s in scat[k - 2]:
                s.wait()
        if k >= 3:
            cw[k - 3].wait()
        for d in loads[k]:
            d.wait()
        if k + 1 < NCHUNK:
            loads[k + 1] = fire_loads(k + 1)
        cw[k] = compute(k)
        scat[k] = fire_scatters(k)
    for s in scat[NCHUNK - 2]:
        s.wait()
    for s in scat[NCHUNK - 1]:
        s.wait()
    for k in range(NCHUNK - 3, NCHUNK):
        cw[k].wait()

    plsc.subcore_barrier()
    pltpu.sync_copy(s1_sh.at[pl.ds(sid * RPT, RPT)],
                    s1p_h.at[cid, pl.ds(sid * RPT, RPT)])


def _k3_body(s1p_h, azp_h, scl_h, srn_h, a_v, b_v, azp_v, scl_v, srn_v, sem):
    cid = lax.axis_index("c")
    sid = lax.axis_index("s")
    w = cid * NS + sid
    last = w == NW - 1
    base = pl.multiple_of(w * RW, 16)
    lane = lax.iota(_i32, L)

    for d in [pltpu.async_copy(s1p_h.at[0, pl.ds(base, RW)], a_v, sem),
              pltpu.async_copy(s1p_h.at[1, pl.ds(base, RW)], b_v, sem)]:
        d.wait()

    def _gg(ref, col, rowi):
        return plsc.load_gather(ref, [rowi, jnp.full((L,), col, _i32)])

    def _grp(gi, _):
        rowi = gi * L + lane
        cnt = _gg(a_v, 14, rowi) + _gg(b_v, 14, rowi)
        scale = 1.0 / jnp.maximum(cnt, 1.0)
        rs = _gg(a_v, 15, rowi) + _gg(b_v, 15, rowi)
        scl_v[pl.ds(gi * L, L)] = scale
        srn_v[pl.ds(gi * L, L)] = scale * rs
        for jp in range(NP):
            ev = (_gg(a_v, 2 * jp, rowi) + _gg(b_v, 2 * jp, rowi)) * scale
            od = (_gg(a_v, 2 * jp + 1, rowi) + _gg(b_v, 2 * jp + 1,
                                                   rowi)) * scale
            word = (lax.shift_right_logical(_bf16_round_bits(ev), 16)
                    | (_bf16_round_bits(od) & jnp.int32(-65536)))
            plsc.store_scatter(azp_v, [jnp.full((L,), jp, _i32), rowi], word)
        return 0
    lax.fori_loop(0, RW // L, _grp, 0)

    @pl.when(jnp.logical_not(last))
    def _full():
        outs = [pltpu.async_copy(azp_v.at[j], azp_h.at[j, pl.ds(base, RW)],
                                 sem) for j in range(NP)]
        outs.append(pltpu.async_copy(scl_v, scl_h.at[pl.ds(base, RW)], sem))
        outs.append(pltpu.async_copy(srn_v, srn_h.at[pl.ds(base, RW)], sem))
        for d in outs:
            d.wait()

    @pl.when(last)
    def _part():
        outs = [pltpu.async_copy(azp_v.at[j, pl.ds(0, RWL)],
                                 azp_h.at[j, pl.ds(base, RWL)], sem)
                for j in range(NP)]
        outs.append(pltpu.async_copy(scl_v.at[pl.ds(0, RWL)],
                                     scl_h.at[pl.ds(base, RWL)], sem))
        outs.append(pltpu.async_copy(srn_v.at[pl.ds(0, RWL)],
                                     srn_h.at[pl.ds(base, RWL)], sem))
        for d in outs:
            d.wait()


def _pass2_body(ei_h, cin_h, azp_h,
                s2p_h,
                azp_v, src_v, dst2_v, c_v, zs_v, lsem, ssem, s2_sh):
    cid = lax.axis_index("c")
    sid = lax.axis_index("s")
    w = cid * NS + sid
    lane = lax.iota(_i32, L)

    stage = [pltpu.async_copy(azp_h.at[j], azp_v.at[pl.ds(j * N, N)],
                              lsem[2])
              for j in range(NP)]
    _zero_acc(zs_v[0], s2_sh, sid, F2)
    for d in stage:
        d.wait()

    def fire_loads(k):
        b = k % 3
        base = pl.multiple_of(w * EPW + CBASE[k], 16)
        ds = [pltpu.async_copy(ei_h.at[0, pl.ds(base, CH)], src_v[b], lsem[b]),
              pltpu.async_copy(cin_h.at[pl.ds(base, CH)], c_v[b], lsem[b])]
        for j in range(NSTR):
            ds.append(pltpu.async_copy(
                ei_h.at[1, pl.ds(pl.multiple_of(base + j * 128, 16), 128)],
                dst2_v[b].at[j], lsem[b]))
        return ds

    def compute(k):
        b, p = k % 3, k % 2
        tail = k == NCHUNK - 1

        def _g(g, _):
            s16 = src_v[b][pl.ds(g * L, L)]
            c16 = c_v[b][pl.ds(g * L, L)]
            rowi = g * L + lane
            if tail:
                c16 = jnp.where(rowi >= TAIL_FROM, c16, 0.0)
            for jp in range(NP):
                wrd = plsc.load_gather(azp_v, [s16 + (jp * N)])
                lo, hi = _unpack_cols(wrd)
                plsc.store_scatter(zs_v[p], [rowi, jnp.full((L,), 2 * jp,
                                                            _i32)], lo * c16)
                plsc.store_scatter(zs_v[p], [rowi, jnp.full((L,), 2 * jp + 1,
                                                            _i32)], hi * c16)
            return 0
        lax.fori_loop(0, NG, _g, 0)

    def fire_scatters(k):
        b, p = k % 3, k % 2
        return [pltpu.async_copy(zs_v[p].at[pl.ds(j * 128, 128)],
                                 s2_sh.at[dst2_v[b].at[j]], ssem[p], add=True)
                for j in range(NSTR)]

    loads = {0: fire_loads(0)}
    scat = {}
    for k in range(NCHUNK):
        if k >= 2:
            for s in scat[k - 2]:
                s.wait()
        for d in loads[k]:
            d.wait()
        if k + 1 < NCHUNK:
            loads[k + 1] = fire_loads(k + 1)
        compute(k)
        scat[k] = fire_scatters(k)
    for s in scat[NCHUNK - 2]:
        s.wait()
    for s in scat[NCHUNK - 1]:
        s.wait()

    plsc.subcore_barrier()
    pltpu.sync_copy(s2_sh.at[pl.ds(sid * RPT, RPT)],
                    s2p_h.at[cid, pl.ds(sid * RPT, RPT)])


def _k5_body(s2p_h, scl_h, srn_h, rb_h, out_h,
             a_v, b_v, scl_v, srn_v, rb_v, out_v, sem):
    cid = lax.axis_index("c")
    sid = lax.axis_index("s")
    w = cid * NS + sid
    last = w == NW - 1
    base = pl.multiple_of(w * RW, 16)
    lane = lax.iota(_i32, L)

    ins = [pltpu.async_copy(s2p_h.at[0, pl.ds(base, RW)], a_v, sem),
           pltpu.async_copy(s2p_h.at[1, pl.ds(base, RW)], b_v, sem),
           pltpu.async_copy(rb_h, rb_v, sem)]

    @pl.when(jnp.logical_not(last))
    def _ldf():
        for d in [pltpu.async_copy(scl_h.at[pl.ds(base, RW)], scl_v, sem),
                  pltpu.async_copy(srn_h.at[pl.ds(base, RW)], srn_v, sem)]:
            d.wait()

    @pl.when(last)
    def _ldp():
        for d in [pltpu.async_copy(scl_h.at[pl.ds(base, RWL)],
                                   scl_v.at[pl.ds(0, RWL)], sem),
                  pltpu.async_copy(srn_h.at[pl.ds(base, RWL)],
                                   srn_v.at[pl.ds(0, RWL)], sem)]:
            d.wait()
    for d in ins:
        d.wait()

    zero16 = jnp.zeros((L,), _i32)
    rjs = [plsc.load_gather(rb_v, [zero16, jnp.full((L,), j, _i32)])
           for j in range(C)]
    bcs = [plsc.load_gather(rb_v, [jnp.full((L,), 1, _i32),
                                   jnp.full((L,), j, _i32)])
           for j in range(C)]

    def _gg(ref, col, rowi):
        return plsc.load_gather(ref, [rowi, jnp.full((L,), col, _i32)])

    def _grp(gi, _):
        rowi = gi * L + lane
        scale = scl_v[pl.ds(gi * L, L)]
        srn = srn_v[pl.ds(gi * L, L)]
        for j in range(C):
            col = ((_gg(a_v, j, rowi) + _gg(b_v, j, rowi)) * scale
                   + srn * rjs[j] + bcs[j])
            plsc.store_scatter(out_v, [rowi, jnp.full((L,), j, _i32)], col)
        return 0
    lax.fori_loop(0, RW // L, _grp, 0)

    @pl.when(jnp.logical_not(last))
    def _stf():
        pltpu.sync_copy(out_v, out_h.at[pl.ds(base, RW)])

    @pl.when(last)
    def _stp():
        pltpu.sync_copy(out_v.at[pl.ds(0, RWL)], out_h.at[pl.ds(base, RWL)])


def kernel(features, edge_index, edge_weight, node_id, alpha,
           W0, b0, W1, b1, Wout, bout):
    ew_f = edge_weight.reshape(E)
    al_p = jnp.concatenate([alpha[:, 0], jnp.zeros((ALP - GENE - 2,), _f32)])

    zp, rb = pl.pallas_call(
        _k1_body,
        out_shape=[jax.ShapeDtypeStruct((NP, N), _i32),
                   jax.ShapeDtypeStruct((2, C), _f32)],
    )(features, W0, W1, Wout, b0[None, :], b1[None, :], bout[None, :])

    mesh = plsc.VectorSubcoreMesh(core_axis_name="c", subcore_axis_name="s")
    params = pltpu.CompilerParams(needs_layout_passes=False,
                                  use_tc_tiling_on_sc=False)

    pass1 = pl.kernel(
        _pass1_body,
        out_type=[jax.ShapeDtypeStruct((NC, NPAD, F1), _f32),
                  jax.ShapeDtypeStruct((E,), _f32)],
        mesh=mesh,
        compiler_params=params,
        scratch_types=[pltpu.VMEM((N,), _i32),
                       pltpu.VMEM((ALP,), _f32),
                       pltpu.VMEM((NP * N,), _i32),
                       [pltpu.VMEM((CH,), _i32)] * 3,
                       [pltpu.VMEM((NSTR, 128), _i32)] * 3,
                       [pltpu.VMEM((CH,), _f32)] * 3,
                       [pltpu.VMEM((CH,), _f32)] * 3,
                       [pltpu.VMEM((CH, F1), _f32)] * 2,
                       pltpu.VMEM_SHARED((NPAD, F1), _f32),
                       [pltpu.SemaphoreType.DMA] * 3,
                       [pltpu.SemaphoreType.DMA] * 2,
                       [pltpu.SemaphoreType.DMA] * 3],
    )
    s1p, cedge = pass1(edge_index, ew_f, node_id, al_p, zp)

    k3 = pl.kernel(
        _k3_body,
        out_type=[jax.ShapeDtypeStruct((NP, N), _i32),
                  jax.ShapeDtypeStruct((N,), _f32),
                  jax.ShapeDtypeStruct((N,), _f32)],
        mesh=mesh,
        compiler_params=params,
        scratch_types=[pltpu.VMEM((RW, F1), _f32),
                       pltpu.VMEM((RW, F1), _f32),
                       pltpu.VMEM((NP, RW), _i32),
                       pltpu.VMEM((RW,), _f32),
                       pltpu.VMEM((RW,), _f32),
                       pltpu.SemaphoreType.DMA],
    )
    azp, scl, srn = k3(s1p)

    pass2 = pl.kernel(
        _pass2_body,
        out_type=[jax.ShapeDtypeStruct((NC, NPAD, F2), _f32)],
        mesh=mesh,
        compiler_params=params,
        scratch_types=[pltpu.VMEM((NP * N,), _i32),
                       [pltpu.VMEM((CH,), _i32)] * 3,
                       [pltpu.VMEM((NSTR, 128), _i32)] * 3,
                       [pltpu.VMEM((CH,), _f32)] * 3,
                       [pltpu.VMEM((CH, F2), _f32)] * 2,
                       [pltpu.SemaphoreType.DMA] * 3,
                       [pltpu.SemaphoreType.DMA] * 2,
                       pltpu.VMEM_SHARED((NPAD, F2), _f32)],
    )
    (s2p,) = pass2(edge_index, cedge, azp)

    k5 = pl.kernel(
        _k5_body,
        out_type=[jax.ShapeDtypeStruct((N, C), _f32)],
        mesh=mesh,
        compiler_params=params,
        scratch_types=[pltpu.VMEM((RW, F1), _f32),
                       pltpu.VMEM((RW, F1), _f32),
                       pltpu.VMEM((RW,), _f32),
                       pltpu.VMEM((RW,), _f32),
                       pltpu.VMEM((2, C), _f32),
                       pltpu.VMEM((RW, C), _f32),
                       pltpu.SemaphoreType.DMA],
    )
    (out,) = k5(s2p, scl, srn, rb)
    return out


# speedup vs baseline: 87.5918x; 1.2596x over previous
import jax
import jax.numpy as jnp
from jax import lax
from jax.experimental import pallas as pl
from jax.experimental.pallas import tpu as pltpu
from jax.experimental.pallas import tpu_sc as plsc

N = 10000
NPAD = 10240
E = 320000
D = 128
C = 10
NP = 5
F1 = 16
F2 = 16
GENE = 5000
ALP = 5008
NC, NS, L = 2, 16, 16
NW = NC * NS
EPW = E // NW
CH = 1024
NCHUNK = 10
CBASE = [0, 1024, 2048, 3072, 4096, 5120, 6144, 7168, 8192, 8976]
TAIL_FROM = 240
NG = CH // L
NSTR = CH // 128
RPT = NPAD // NS
RW = NPAD // NW
RWL = N - (NW - 1) * RW

_f32 = jnp.float32
_i32 = jnp.int32


def _pack_bf16_pairs_t(xt):
    b = lax.bitcast_convert_type(xt, _i32)
    r = b + jnp.int32(0x7FFF) + (lax.shift_right_logical(b, 16) & 1)
    h = lax.shift_right_logical(r, 16)
    rows = [h[2 * j:2 * j + 1, :]
            | lax.shift_left(h[2 * j + 1:2 * j + 2, :], 16)
            for j in range(NP)]
    return jnp.concatenate(rows, axis=0)


def _k1_body(x_ref, w0_ref, w1_ref, wout_ref, b0_ref, b1_ref, bout_ref,
             zp_ref, rb_ref):
    wc = jnp.dot(wout_ref[...], w1_ref[...], preferred_element_type=_f32)
    wd = jnp.dot(wc, w0_ref[...], preferred_element_type=_f32)
    zt = lax.dot_general(wd, x_ref[...], (((1,), (1,)), ((), ())),
                         preferred_element_type=_f32)
    zp_ref[...] = _pack_bf16_pairs_t(zt)
    r = jnp.dot(b0_ref[...], wc.T, preferred_element_type=_f32)
    bc = jnp.dot(b1_ref[...], wout_ref[...].T,
                 preferred_element_type=_f32) + bout_ref[...]
    rb_ref[...] = jnp.concatenate([r, bc], axis=0)


def _zero_acc(zs, sh, sid, width):
    lane = lax.iota(_i32, L)
    zeros16 = jnp.zeros((L,), _f32)

    @plsc.parallel_loop(0, RPT // L, unroll=2)
    def _z(g):
        rowi = g * L + lane
        for j in range(width):
            plsc.store_scatter(zs, [rowi, jnp.full((L,), j, _i32)], zeros16)
    pltpu.sync_copy(zs.at[pl.ds(0, RPT)], sh.at[pl.ds(sid * RPT, RPT)])
    plsc.subcore_barrier()


def _unpack_cols(w):
    lo = plsc.bitcast(lax.shift_left(w, 16), _f32)
    hi = plsc.bitcast(w & jnp.int32(-65536), _f32)
    return lo, hi


def _bf16_round_bits(x):
    b = plsc.bitcast(x, _i32)
    return b + jnp.int32(0x7FFF) + (lax.shift_right_logical(b, 16) & 1)


def _pass1_body(ei_h, ew_h, nid_h, al_h, zp_h,
                s1p_h, c_h,
                nid_v, al_v, zp_v, src_v, dst2_v, ew_v, c_v, zs_v,
                s1_sh, lsem, ssem, wsem):
    cid = lax.axis_index("c")
    sid = lax.axis_index("s")
    w = cid * NS + sid
    lane = lax.iota(_i32, L)

    stage = [pltpu.async_copy(nid_h, nid_v, wsem[0]),
             pltpu.async_copy(al_h, al_v, wsem[0])]
    stage += [pltpu.async_copy(zp_h.at[j], zp_v.at[pl.ds(j * N, N)], wsem[0])
              for j in range(NP)]
    _zero_acc(zs_v[0], s1_sh, sid, F1)
    for d in stage:
        d.wait()

    def fire_loads(k):
        b = k % 3
        base = pl.multiple_of(w * EPW + CBASE[k], 16)
        ds = [pltpu.async_copy(ei_h.at[0, pl.ds(base, CH)], src_v[b], lsem[b]),
              pltpu.async_copy(ew_h.at[pl.ds(base, CH)], ew_v[b], lsem[b])]
        for j in range(NSTR):
            ds.append(pltpu.async_copy(
                ei_h.at[1, pl.ds(pl.multiple_of(base + j * 128, 16), 128)],
                dst2_v[b].at[j], lsem[b]))
        return ds

    def compute(k):
        b, p = k % 3, k % 2
        tail = k == NCHUNK - 1

        @plsc.parallel_loop(0, NG, unroll=2)
        def _g(g):
            s16 = src_v[b][pl.ds(g * L, L)]
            d16 = dst2_v[b][g // 8, pl.ds((g % 8) * L, L)]
            sid16 = plsc.load_gather(nid_v, [s16])
            did16 = plsc.load_gather(nid_v, [d16])
            sg = sid16 >= 0
            dg = did16 >= 0
            idx16 = jnp.full((L,), GENE + 1, _i32)
            idx16 = jnp.where(sg & (~dg), sid16, idx16)
            idx16 = jnp.where(dg & (~sg), did16, idx16)
            idx16 = jnp.where(dg & sg, jnp.full((L,), GENE, _i32), idx16)
            a16 = plsc.load_gather(al_v, [idx16])
            c16 = a16 * ew_v[b][pl.ds(g * L, L)]
            rowi = g * L + lane
            valid = jnp.ones((L,), _f32) if not tail else jnp.where(
                rowi >= TAIL_FROM, 1.0, 0.0)
            if tail:
                c16 = c16 * valid
            c_v[b][pl.ds(g * L, L)] = c16
            for jp in range(NP):
                wrd = plsc.load_gather(zp_v, [s16 + (jp * N)])
                lo, hi = _unpack_cols(wrd)
                plsc.store_scatter(zs_v[p], [rowi, jnp.full((L,), 2 * jp,
                                                            _i32)], lo * c16)
                plsc.store_scatter(zs_v[p], [rowi, jnp.full((L,), 2 * jp + 1,
                                                            _i32)], hi * c16)
            plsc.store_scatter(zs_v[p], [rowi, jnp.full((L,), 14, _i32)],
                               valid)
            plsc.store_scatter(zs_v[p], [rowi, jnp.full((L,), 15, _i32)], c16)
        base = pl.multiple_of(w * EPW + CBASE[k], 16)
        if not tail:
            return pltpu.async_copy(c_v[b], c_h.at[pl.ds(base, CH)], wsem[b])
        return pltpu.async_copy(
            c_v[b].at[pl.ds(TAIL_FROM, CH - TAIL_FROM)],
            c_h.at[pl.ds(pl.multiple_of(base + TAIL_FROM, 16),
                         CH - TAIL_FROM)], wsem[b])

    def fire_scatters(k):
        b, p = k % 3, k % 2
        return [pltpu.async_copy(zs_v[p].at[pl.ds(j * 128, 128)],
                                 s1_sh.at[dst2_v[b].at[j]], ssem[p], add=True)
                for j in range(NSTR)]

    loads = {0: fire_loads(0)}
    scat, cw = {}, {}
    for k in range(NCHUNK):
        if k >= 2:
            for s in scat[k - 2]:
                s.wait()
        if k >= 3:
            cw[k - 3].wait()
        for d in loads[k]:
            d.wait()
        if k + 1 < NCHUNK:
            loads[k + 1] = fire_loads(k + 1)
        cw[k] = compute(k)
        scat[k] = fire_scatters(k)
    for s in scat[NCHUNK - 2]:
        s.wait()
    for s in scat[NCHUNK - 1]:
        s.wait()
    for k in range(NCHUNK - 3, NCHUNK):
        cw[k].wait()

    plsc.subcore_barrier()
    pltpu.sync_copy(s1_sh.at[pl.ds(sid * RPT, RPT)],
                    s1p_h.at[cid, pl.ds(sid * RPT, RPT)])


def _k3_body(s1p_h, azp_h, scl_h, srn_h, a_v, b_v, azp_v, scl_v, srn_v, sem):
    cid = lax.axis_index("c")
    sid = lax.axis_index("s")
    w = cid * NS + sid
    last = w == NW - 1
    base = pl.multiple_of(w * RW, 16)
    lane = lax.iota(_i32, L)

    for d in [pltpu.async_copy(s1p_h.at[0, pl.ds(base, RW)], a_v, sem),
              pltpu.async_copy(s1p_h.at[1, pl.ds(base, RW)], b_v, sem)]:
        d.wait()

    def _gg(ref, col, rowi):
        return plsc.load_gather(ref, [rowi, jnp.full((L,), col, _i32)])

    @plsc.parallel_loop(0, RW // L, unroll=2)
    def _grp(gi):
        rowi = gi * L + lane
        cnt = _gg(a_v, 14, rowi) + _gg(b_v, 14, rowi)
        scale = 1.0 / jnp.maximum(cnt, 1.0)
        rs = _gg(a_v, 15, rowi) + _gg(b_v, 15, rowi)
        scl_v[pl.ds(gi * L, L)] = scale
        srn_v[pl.ds(gi * L, L)] = scale * rs
        for jp in range(NP):
            ev = (_gg(a_v, 2 * jp, rowi) + _gg(b_v, 2 * jp, rowi)) * scale
            od = (_gg(a_v, 2 * jp + 1, rowi) + _gg(b_v, 2 * jp + 1,
                                                   rowi)) * scale
            word = (lax.shift_right_logical(_bf16_round_bits(ev), 16)
                    | (_bf16_round_bits(od) & jnp.int32(-65536)))
            plsc.store_scatter(azp_v, [jnp.full((L,), jp, _i32), rowi], word)

    @pl.when(jnp.logical_not(last))
    def _full():
        outs = [pltpu.async_copy(azp_v.at[j], azp_h.at[j, pl.ds(base, RW)],
                                 sem) for j in range(NP)]
        outs.append(pltpu.async_copy(scl_v, scl_h.at[pl.ds(base, RW)], sem))
        outs.append(pltpu.async_copy(srn_v, srn_h.at[pl.ds(base, RW)], sem))
        for d in outs:
            d.wait()

    @pl.when(last)
    def _part():
        outs = [pltpu.async_copy(azp_v.at[j, pl.ds(0, RWL)],
                                 azp_h.at[j, pl.ds(base, RWL)], sem)
                for j in range(NP)]
        outs.append(pltpu.async_copy(scl_v.at[pl.ds(0, RWL)],
                                     scl_h.at[pl.ds(base, RWL)], sem))
        outs.append(pltpu.async_copy(srn_v.at[pl.ds(0, RWL)],
                                     srn_h.at[pl.ds(base, RWL)], sem))
        for d in outs:
            d.wait()


def _pass2_body(ei_h, cin_h, azp_h,
                s2p_h,
                azp_v, src_v, dst2_v, c_v, zs_v, lsem, ssem, s2_sh):
    cid = lax.axis_index("c")
    sid = lax.axis_index("s")
    w = cid * NS + sid
    lane = lax.iota(_i32, L)

    stage = [pltpu.async_copy(azp_h.at[j], azp_v.at[pl.ds(j * N, N)],
                              lsem[2])
              for j in range(NP)]
    _zero_acc(zs_v[0], s2_sh, sid, F2)
    for d in stage:
        d.wait()

    def fire_loads(k):
        b = k % 3
        base = pl.multiple_of(w * EPW + CBASE[k], 16)
        ds = [pltpu.async_copy(ei_h.at[0, pl.ds(base, CH)], src_v[b], lsem[b]),
              pltpu.async_copy(cin_h.at[pl.ds(base, CH)], c_v[b], lsem[b])]
        for j in range(NSTR):
            ds.append(pltpu.async_copy(
                ei_h.at[1, pl.ds(pl.multiple_of(base + j * 128, 16), 128)],
                dst2_v[b].at[j], lsem[b]))
        return ds

    def compute(k):
        b, p = k % 3, k % 2
        tail = k == NCHUNK - 1

        @plsc.parallel_loop(0, NG, unroll=2)
        def _g(g):
            s16 = src_v[b][pl.ds(g * L, L)]
            c16 = c_v[b][pl.ds(g * L, L)]
            rowi = g * L + lane
            if tail:
                c16 = jnp.where(rowi >= TAIL_FROM, c16, 0.0)
            for jp in range(NP):
                wrd = plsc.load_gather(azp_v, [s16 + (jp * N)])
                lo, hi = _unpack_cols(wrd)
                plsc.store_scatter(zs_v[p], [rowi, jnp.full((L,), 2 * jp,
                                                            _i32)], lo * c16)
                plsc.store_scatter(zs_v[p], [rowi, jnp.full((L,), 2 * jp + 1,
                                                            _i32)], hi * c16)

    def fire_scatters(k):
        b, p = k % 3, k % 2
        return [pltpu.async_copy(zs_v[p].at[pl.ds(j * 128, 128)],
                                 s2_sh.at[dst2_v[b].at[j]], ssem[p], add=True)
                for j in range(NSTR)]

    loads = {0: fire_loads(0)}
    scat = {}
    for k in range(NCHUNK):
        if k >= 2:
            for s in scat[k - 2]:
                s.wait()
        for d in loads[k]:
            d.wait()
        if k + 1 < NCHUNK:
            loads[k + 1] = fire_loads(k + 1)
        compute(k)
        scat[k] = fire_scatters(k)
    for s in scat[NCHUNK - 2]:
        s.wait()
    for s in scat[NCHUNK - 1]:
        s.wait()

    plsc.subcore_barrier()
    pltpu.sync_copy(s2_sh.at[pl.ds(sid * RPT, RPT)],
                    s2p_h.at[cid, pl.ds(sid * RPT, RPT)])


def _k5_body(s2p_h, scl_h, srn_h, rb_h, out_h,
             a_v, b_v, scl_v, srn_v, rb_v, out_v, sem):
    cid = lax.axis_index("c")
    sid = lax.axis_index("s")
    w = cid * NS + sid
    last = w == NW - 1
    base = pl.multiple_of(w * RW, 16)
    lane = lax.iota(_i32, L)

    ins = [pltpu.async_copy(s2p_h.at[0, pl.ds(base, RW)], a_v, sem),
           pltpu.async_copy(s2p_h.at[1, pl.ds(base, RW)], b_v, sem),
           pltpu.async_copy(rb_h, rb_v, sem)]

    @pl.when(jnp.logical_not(last))
    def _ldf():
        for d in [pltpu.async_copy(scl_h.at[pl.ds(base, RW)], scl_v, sem),
                  pltpu.async_copy(srn_h.at[pl.ds(base, RW)], srn_v, sem)]:
            d.wait()

    @pl.when(last)
    def _ldp():
        for d in [pltpu.async_copy(scl_h.at[pl.ds(base, RWL)],
                                   scl_v.at[pl.ds(0, RWL)], sem),
                  pltpu.async_copy(srn_h.at[pl.ds(base, RWL)],
                                   srn_v.at[pl.ds(0, RWL)], sem)]:
            d.wait()
    for d in ins:
        d.wait()

    zero16 = jnp.zeros((L,), _i32)
    rjs = [plsc.load_gather(rb_v, [zero16, jnp.full((L,), j, _i32)])
           for j in range(C)]
    bcs = [plsc.load_gather(rb_v, [jnp.full((L,), 1, _i32),
                                   jnp.full((L,), j, _i32)])
           for j in range(C)]

    def _gg(ref, col, rowi):
        return plsc.load_gather(ref, [rowi, jnp.full((L,), col, _i32)])

    @plsc.parallel_loop(0, RW // L, unroll=2)
    def _grp(gi):
        rowi = gi * L + lane
        scale = scl_v[pl.ds(gi * L, L)]
        srn = srn_v[pl.ds(gi * L, L)]
        for j in range(C):
            col = ((_gg(a_v, j, rowi) + _gg(b_v, j, rowi)) * scale
                   + srn * rjs[j] + bcs[j])
            plsc.store_scatter(out_v, [rowi, jnp.full((L,), j, _i32)], col)

    @pl.when(jnp.logical_not(last))
    def _stf():
        pltpu.sync_copy(out_v, out_h.at[pl.ds(base, RW)])

    @pl.when(last)
    def _stp():
        pltpu.sync_copy(out_v.at[pl.ds(0, RWL)], out_h.at[pl.ds(base, RWL)])


def kernel(features, edge_index, edge_weight, node_id, alpha,
           W0, b0, W1, b1, Wout, bout):
    ew_f = edge_weight.reshape(E)
    al_p = jnp.concatenate([alpha[:, 0], jnp.zeros((ALP - GENE - 2,), _f32)])

    zp, rb = pl.pallas_call(
        _k1_body,
        out_shape=[jax.ShapeDtypeStruct((NP, N), _i32),
                   jax.ShapeDtypeStruct((2, C), _f32)],
    )(features, W0, W1, Wout, b0[None, :], b1[None, :], bout[None, :])

    mesh = plsc.VectorSubcoreMesh(core_axis_name="c", subcore_axis_name="s")
    params = pltpu.CompilerParams(needs_layout_passes=False,
                                  use_tc_tiling_on_sc=False)

    pass1 = pl.kernel(
        _pass1_body,
        out_type=[jax.ShapeDtypeStruct((NC, NPAD, F1), _f32),
                  jax.ShapeDtypeStruct((E,), _f32)],
        mesh=mesh,
        compiler_params=params,
        scratch_types=[pltpu.VMEM((N,), _i32),
                       pltpu.VMEM((ALP,), _f32),
                       pltpu.VMEM((NP * N,), _i32),
                       [pltpu.VMEM((CH,), _i32)] * 3,
                       [pltpu.VMEM((NSTR, 128), _i32)] * 3,
                       [pltpu.VMEM((CH,), _f32)] * 3,
                       [pltpu.VMEM((CH,), _f32)] * 3,
                       [pltpu.VMEM((CH, F1), _f32)] * 2,
                       pltpu.VMEM_SHARED((NPAD, F1), _f32),
                       [pltpu.SemaphoreType.DMA] * 3,
                       [pltpu.SemaphoreType.DMA] * 2,
                       [pltpu.SemaphoreType.DMA] * 3],
    )
    s1p, cedge = pass1(edge_index, ew_f, node_id, al_p, zp)

    k3 = pl.kernel(
        _k3_body,
        out_type=[jax.ShapeDtypeStruct((NP, N), _i32),
                  jax.ShapeDtypeStruct((N,), _f32),
                  jax.ShapeDtypeStruct((N,), _f32)],
        mesh=mesh,
        compiler_params=params,
        scratch_types=[pltpu.VMEM((RW, F1), _f32),
                       pltpu.VMEM((RW, F1), _f32),
                       pltpu.VMEM((NP, RW), _i32),
                       pltpu.VMEM((RW,), _f32),
                       pltpu.VMEM((RW,), _f32),
                       pltpu.SemaphoreType.DMA],
    )
    azp, scl, srn = k3(s1p)

    pass2 = pl.kernel(
        _pass2_body,
        out_type=[jax.ShapeDtypeStruct((NC, NPAD, F2), _f32)],
        mesh=mesh,
        compiler_params=params,
        scratch_types=[pltpu.VMEM((NP * N,), _i32),
                       [pltpu.VMEM((CH,), _i32)] * 3,
                       [pltpu.VMEM((NSTR, 128), _i32)] * 3,
                       [pltpu.VMEM((CH,), _f32)] * 3,
                       [pltpu.VMEM((CH, F2), _f32)] * 2,
                       [pltpu.SemaphoreType.DMA] * 3,
                       [pltpu.SemaphoreType.DMA] * 2,
                       pltpu.VMEM_SHARED((NPAD, F2), _f32)],
    )
    (s2p,) = pass2(edge_index, cedge, azp)

    k5 = pl.kernel(
        _k5_body,
        out_type=[jax.ShapeDtypeStruct((N, C), _f32)],
        mesh=mesh,
        compiler_params=params,
        scratch_types=[pltpu.VMEM((RW, F1), _f32),
                       pltpu.VMEM((RW, F1), _f32),
                       pltpu.VMEM((RW,), _f32),
                       pltpu.VMEM((RW,), _f32),
                       pltpu.VMEM((2, C), _f32),
                       pltpu.VMEM((RW, C), _f32),
                       pltpu.SemaphoreType.DMA],
    )
    (out,) = k5(s2p, scl, srn, rb)
    return out
